# Initial kernel scaffold; baseline (speedup 1.0000x reference)
import functools

import jax
import jax.numpy as jnp
from jax import lax
from jax.experimental import pallas as pl
from jax.experimental.pallas import tpu as pltpu
from jax.experimental.pallas import tpu_sc as plsc

_N = 10000
_E = 160000
_NW = 32
_EP = _E // _NW
_EPP = 5120
_NCH = _EPP // 128
_NP = 10016
_RPT = _NP // 16
_DUMMY = _N
_HISTW = 16
_HISTR = 640


def _mesh():
    return plsc.VectorSubcoreMesh(core_axis_name="c", subcore_axis_name="s")


@functools.lru_cache(maxsize=None)
def _sc_pre():
    zi = jnp.zeros((16,), jnp.int32)
    zf = jnp.zeros((16,), jnp.float32)
    ones = jnp.ones((16,), jnp.float32)

    def body(ei, rowpad, colpad, hist, row_v, col_v, hist_v):
        c = lax.axis_index("c")
        s = lax.axis_index("s")
        wid = c * 16 + s
        base = wid * _EP

        def zh(i, carry):
            hist_v[i] = zf
            return carry
        lax.fori_loop(0, _HISTR, zh, 0)

        pltpu.sync_copy(ei.at[0, pl.ds(base, _EP)], row_v.at[pl.ds(0, _EP)])
        pltpu.sync_copy(ei.at[1, pl.ds(base, _EP)], col_v.at[pl.ds(0, _EP)])
        for k in range(7):
            row_v[pl.ds(_EP + 16 * k, 16)] = zi
            col_v[pl.ds(_EP + 16 * k, 16)] = zi
        row_v[pl.ds(_EPP - 16, 16)] = zi
        col_v[pl.ds(_EPP - 16, 16)] = zi

        def eb(j, carry):
            off = j * 16
            r = row_v[pl.ds(off, 16)]
            cc = col_v[pl.ds(off, 16)]
            self_ = r == cc
            col_v[pl.ds(off, 16)] = jnp.where(self_, _DUMMY, cc)
            di = jnp.where(self_, _HISTR * _HISTW - 1, r)
            plsc.addupdate_scatter(
                hist_v,
                [lax.shift_right_logical(di, 4), lax.bitwise_and(di, 15)],
                ones,
            )
            return carry
        lax.fori_loop(0, _EPP // 16, eb, 0)

        pltpu.sync_copy(row_v, rowpad.at[pl.ds(wid * _EPP, _EPP)])
        pltpu.sync_copy(col_v, colpad.at[pl.ds(wid * _EPP, _EPP)])
        pltpu.sync_copy(hist_v, hist.at[wid])

    return pl.kernel(
        body,
        out_type=(
            jax.ShapeDtypeStruct((_NW * _EPP,), jnp.int32),
            jax.ShapeDtypeStruct((_NW * _EPP,), jnp.int32),
            jax.ShapeDtypeStruct((_NW, _HISTR, _HISTW), jnp.float32),
        ),
        mesh=_mesh(),
        scratch_types=[
            pltpu.VMEM((_EPP,), jnp.int32),
            pltpu.VMEM((_EPP,), jnp.int32),
            pltpu.VMEM((_HISTR, _HISTW), jnp.float32),
        ],
    )


@functools.lru_cache(maxsize=None)
def _sc_edge(D):
    zf = jnp.zeros((16,), jnp.float32)

    def body(z, rowi, coli, t_out, ri_v, ci_v, g0, g1, zb_v, t_sh, sem0, sem1):
        c = lax.axis_index("c")
        s = lax.axis_index("s")
        wid = c * 16 + s

        def zb(i, carry):
            for k in range(D // 16):
                zb_v[i, pl.ds(16 * k, 16)] = zf
            return carry
        lax.fori_loop(0, _RPT, zb, 0)
        pltpu.sync_copy(zb_v, t_sh.at[pl.ds(s * _RPT, _RPT)])

        pltpu.sync_copy(rowi.at[pl.ds(wid * _NCH, _NCH)], ri_v)
        pltpu.sync_copy(coli.at[pl.ds(wid * _NCH, _NCH)], ci_v)
        plsc.subcore_barrier()

        def start_g(j, buf, sem):
            pltpu.async_copy(z.at[ri_v.at[j]], buf, sem)

        def wait_g(buf, sem):
            pltpu.make_async_copy(z.at[ri_v.at[0]], buf, sem).wait()

        start_g(0, g0, sem0)

        def lb(k, carry):
            j0 = 2 * k
            j1 = j0 + 1
            wait_g(g0, sem0)
            start_g(j1, g1, sem1)
            pltpu.sync_copy(g0, t_sh.at[ci_v.at[j0]], add=True)
            wait_g(g1, sem1)

            @pl.when(j1 + 1 < _NCH)
            def _():
                start_g(j1 + 1, g0, sem0)

            pltpu.sync_copy(g1, t_sh.at[ci_v.at[j1]], add=True)
            return carry
        lax.fori_loop(0, _NCH // 2, lb, 0)

        plsc.subcore_barrier()
        pltpu.sync_copy(t_sh.at[pl.ds(s * _RPT, _RPT)], zb_v)
        pltpu.sync_copy(zb_v, t_out.at[c, pl.ds(s * _RPT, _RPT)])

    return pl.kernel(
        body,
        out_type=jax.ShapeDtypeStruct((2, _NP, D), jnp.float32),
        mesh=_mesh(),
        scratch_types=[
            pltpu.VMEM((_NCH, 128), jnp.int32),
            pltpu.VMEM((_NCH, 128), jnp.int32),
            pltpu.VMEM((128, D), jnp.float32),
            pltpu.VMEM((128, D), jnp.float32),
            pltpu.VMEM((_RPT, D), jnp.float32),
            pltpu.VMEM_SHARED((_NP, D), jnp.float32),
            pltpu.SemaphoreType.DMA,
            pltpu.SemaphoreType.DMA,
        ],
    )


def _tc_dis_body(hist_ref, dis_ref):
    deg = jnp.sum(hist_ref[...], axis=0)
    safe = jnp.where(deg > 0, deg, 1.0)
    dis_ref[...] = jnp.where(deg > 0, lax.rsqrt(safe), 0.0)[None, :]


@functools.lru_cache(maxsize=None)
def _tc_dis():
    return pl.pallas_call(
        _tc_dis_body,
        out_shape=jax.ShapeDtypeStruct((1, _HISTR * _HISTW), jnp.float32),
    )


_BN = 400
_NB = _N // _BN


def _tc_d1_body(x_ref, w10_ref, w11_ref, dis_ref, z0_ref, zs1_ref):
    xb = x_ref[...]
    z0_ref[...] = jnp.dot(xb, w10_ref[...], preferred_element_type=jnp.float32)
    zs1_ref[...] = (
        jnp.dot(xb, w11_ref[...], preferred_element_type=jnp.float32)
        * dis_ref[...]
    )


@functools.lru_cache(maxsize=None)
def _tc_d1(din, dh):
    return pl.pallas_call(
        _tc_d1_body,
        grid=(_NB,),
        in_specs=[
            pl.BlockSpec((_BN, din), lambda i: (i, 0)),
            pl.BlockSpec((din, dh), lambda i: (0, 0)),
            pl.BlockSpec((din, dh), lambda i: (0, 0)),
            pl.BlockSpec((_BN, 1), lambda i: (i, 0)),
        ],
        out_specs=[
            pl.BlockSpec((_BN, dh), lambda i: (i, 0)),
            pl.BlockSpec((_BN, dh), lambda i: (i, 0)),
        ],
        out_shape=[
            jax.ShapeDtypeStruct((_N, dh), jnp.float32),
            jax.ShapeDtypeStruct((_N, dh), jnp.float32),
        ],
    )


def _tc_d2_body(z0_ref, t1_ref, dis_ref, b1_ref, w20_ref, w21_ref,
                z20_ref, zs2_ref):
    t = t1_ref[...]
    dis = dis_ref[...]
    h = jnp.maximum(z0_ref[...] - dis * (t[0] + t[1]) + b1_ref[...], 0.0)
    z20_ref[...] = jnp.dot(h, w20_ref[...], preferred_element_type=jnp.float32)
    zs2_ref[...] = (
        jnp.dot(h, w21_ref[...], preferred_element_type=jnp.float32) * dis
    )


@functools.lru_cache(maxsize=None)
def _tc_d2(dh, do):
    return pl.pallas_call(
        _tc_d2_body,
        grid=(_NB,),
        in_specs=[
            pl.BlockSpec((_BN, dh), lambda i: (i, 0)),
            pl.BlockSpec((2, _BN, dh), lambda i: (0, i, 0)),
            pl.BlockSpec((_BN, 1), lambda i: (i, 0)),
            pl.BlockSpec((1, dh), lambda i: (0, 0)),
            pl.BlockSpec((dh, do), lambda i: (0, 0)),
            pl.BlockSpec((dh, do), lambda i: (0, 0)),
        ],
        out_specs=[
            pl.BlockSpec((_BN, do), lambda i: (i, 0)),
            pl.BlockSpec((_BN, do), lambda i: (i, 0)),
        ],
        out_shape=[
            jax.ShapeDtypeStruct((_N, do), jnp.float32),
            jax.ShapeDtypeStruct((_N, do), jnp.float32),
        ],
    )


def _tc_fin_body(z20_ref, t2_ref, dis_ref, b2_ref, out_ref):
    t = t2_ref[...]
    o = z20_ref[...] - dis_ref[...] * (t[0] + t[1]) + b2_ref[...]
    m = jnp.max(o, axis=1, keepdims=True)
    lse = jnp.log(jnp.sum(jnp.exp(o - m), axis=1, keepdims=True))
    out_ref[...] = o - m - lse


@functools.lru_cache(maxsize=None)
def _tc_fin(do):
    return pl.pallas_call(
        _tc_fin_body,
        grid=(_NB,),
        in_specs=[
            pl.BlockSpec((_BN, do), lambda i: (i, 0)),
            pl.BlockSpec((2, _BN, do), lambda i: (0, i, 0)),
            pl.BlockSpec((_BN, 1), lambda i: (i, 0)),
            pl.BlockSpec((1, do), lambda i: (0, 0)),
        ],
        out_specs=pl.BlockSpec((_BN, do), lambda i: (i, 0)),
        out_shape=jax.ShapeDtypeStruct((_N, do), jnp.float32),
    )


def kernel(x, edge_index, W10, W11, b1, W20, W21, b2):
    din = x.shape[1]
    dh = W10.shape[1]
    do = W20.shape[1]

    rowpad, colpad, hist = _sc_pre()(edge_index)
    dis_row = _tc_dis()(hist.reshape(_NW, _HISTR * _HISTW))
    dis = dis_row.reshape(_HISTR * _HISTW, 1)

    z0, zs1 = _tc_d1(din, dh)(x, W10, W11, dis)
    row2d = rowpad.reshape(_NW * _NCH, 128)
    col2d = colpad.reshape(_NW * _NCH, 128)
    t1 = _sc_edge(dh)(zs1, row2d, col2d)
    z20, zs2 = _tc_d2(dh, do)(z0, t1, dis, b1.reshape(1, dh), W20, W21)
    t2 = _sc_edge(do)(zs2, row2d, col2d)
    return _tc_fin(do)(z20, t2, dis, b2.reshape(1, do))


# speedup vs baseline: 16.2230x; 16.2230x over previous
import functools

import jax
import jax.numpy as jnp
from jax import lax
from jax.experimental import pallas as pl
from jax.experimental.pallas import tpu as pltpu
from jax.experimental.pallas import tpu_sc as plsc

_N = 10000
_E = 160000
_NW = 32
_EP = _E // _NW
_EPP = 5120
_NCH = _EPP // 128
_NP = 10112
_RPT = _NP // 16
_DUMMY = _N
_HISTW = 16
_HISTR = 640


def _mesh():
    return plsc.VectorSubcoreMesh(core_axis_name="c", subcore_axis_name="s")


@functools.lru_cache(maxsize=None)
def _sc_pre():
    def body(ei, rowpad, colpad, hist, row_v, col_v, hist_v):
        c = lax.axis_index("c")
        s = lax.axis_index("s")
        wid = c * 16 + s
        base = wid * _EP
        iota16 = lax.iota(jnp.int32, 16)
        zi = iota16 * 0
        zf = zi.astype(jnp.float32)
        ones = zf + 1.0

        def zh(i, carry):
            hist_v[pl.ds(i * 16, 16)] = zf
            return carry
        lax.fori_loop(0, _HISTR, zh, 0)

        pltpu.sync_copy(ei.at[pl.ds(base, _EP)], row_v.at[pl.ds(0, _EP)])
        pltpu.sync_copy(ei.at[pl.ds(_E + base, _EP)], col_v.at[pl.ds(0, _EP)])
        for k in range(7):
            row_v[pl.ds(_EP + 16 * k, 16)] = zi
            col_v[pl.ds(_EP + 16 * k, 16)] = zi
        row_v[pl.ds(_EPP - 16, 16)] = zi
        col_v[pl.ds(_EPP - 16, 16)] = zi

        def eb(j, carry):
            off = j * 16
            r = row_v[pl.ds(off, 16)]
            cc = col_v[pl.ds(off, 16)]
            self_ = r == cc
            col_v[pl.ds(off, 16)] = jnp.where(self_, _DUMMY, cc)
            di = jnp.where(self_, _HISTR * _HISTW - 1, r)
            plsc.addupdate_scatter(hist_v, [di], ones)
            return carry
        lax.fori_loop(0, _EPP // 16, eb, 0)

        pltpu.sync_copy(row_v, rowpad.at[pl.ds(wid * _EPP, _EPP)])
        pltpu.sync_copy(col_v, colpad.at[pl.ds(wid * _EPP, _EPP)])
        pltpu.sync_copy(hist_v, hist.at[wid])

    return pl.kernel(
        body,
        out_type=(
            jax.ShapeDtypeStruct((_NW * _EPP,), jnp.int32),
            jax.ShapeDtypeStruct((_NW * _EPP,), jnp.int32),
            jax.ShapeDtypeStruct((_NW, _HISTR * _HISTW), jnp.float32),
        ),
        mesh=_mesh(),
        scratch_types=[
            pltpu.VMEM((_EPP,), jnp.int32),
            pltpu.VMEM((_EPP,), jnp.int32),
            pltpu.VMEM((_HISTR * _HISTW,), jnp.float32),
        ],
        compiler_params=pltpu.CompilerParams(needs_layout_passes=False),
    )


@functools.lru_cache(maxsize=None)
def _sc_edge(D):
    def body(z, rowi, coli, t_out, ri_v, ci_v, g0, g1, zb_v, t_sh, sem0, sem1):
        c = lax.axis_index("c")
        s = lax.axis_index("s")
        wid = c * 16 + s
        zf = lax.iota(jnp.int32, 16).astype(jnp.float32) * 0.0

        def zb(i, carry):
            for k in range(D // 16):
                zb_v[i, pl.ds(16 * k, 16)] = zf
            return carry
        lax.fori_loop(0, _RPT, zb, 0)
        pltpu.sync_copy(zb_v, t_sh.at[pl.ds(s * _RPT, _RPT)])

        pltpu.sync_copy(rowi.at[pl.ds(wid * _NCH, _NCH)], ri_v)
        pltpu.sync_copy(coli.at[pl.ds(wid * _NCH, _NCH)], ci_v)
        plsc.subcore_barrier()

        def start_g(j, buf, sem):
            pltpu.async_copy(z.at[ri_v.at[j]], buf, sem)

        def wait_g(buf, sem):
            pltpu.make_async_copy(z.at[ri_v.at[0]], buf, sem).wait()

        start_g(0, g0, sem0)

        def lb(k, carry):
            j0 = 2 * k
            j1 = j0 + 1
            wait_g(g0, sem0)
            start_g(j1, g1, sem1)
            pltpu.sync_copy(g0, t_sh.at[ci_v.at[j0]], add=True)
            wait_g(g1, sem1)

            @pl.when(j1 + 1 < _NCH)
            def _():
                start_g(j1 + 1, g0, sem0)

            pltpu.sync_copy(g1, t_sh.at[ci_v.at[j1]], add=True)
            return carry
        lax.fori_loop(0, _NCH // 2, lb, 0)

        plsc.subcore_barrier()
        pltpu.sync_copy(t_sh.at[pl.ds(s * _RPT, _RPT)], zb_v)
        pltpu.sync_copy(zb_v, t_out.at[c, pl.ds(s * _RPT, _RPT)])

    return pl.kernel(
        body,
        out_type=jax.ShapeDtypeStruct((2, _NP, D), jnp.float32),
        mesh=_mesh(),
        scratch_types=[
            pltpu.VMEM((_NCH, 128), jnp.int32),
            pltpu.VMEM((_NCH, 128), jnp.int32),
            pltpu.VMEM((128, D), jnp.float32),
            pltpu.VMEM((128, D), jnp.float32),
            pltpu.VMEM((_RPT, D), jnp.float32),
            pltpu.VMEM_SHARED((_NP, D), jnp.float32),
            pltpu.SemaphoreType.DMA,
            pltpu.SemaphoreType.DMA,
        ],
        compiler_params=pltpu.CompilerParams(
            needs_layout_passes=False, use_tc_tiling_on_sc=False
        ),
    )


def _tc_dis_body(hist_ref, dis_ref):
    deg = jnp.sum(hist_ref[...], axis=0)
    safe = jnp.where(deg > 0, deg, 1.0)
    dis_ref[...] = jnp.where(deg > 0, lax.rsqrt(safe), 0.0)[None, :]


@functools.lru_cache(maxsize=None)
def _tc_dis():
    return pl.pallas_call(
        _tc_dis_body,
        out_shape=jax.ShapeDtypeStruct((1, _HISTR * _HISTW), jnp.float32),
    )


_BN = 400
_NB = _N // _BN


def _tc_d1_body(x_ref, w10_ref, w11_ref, dis_ref, z0_ref, zs1_ref):
    xb = x_ref[...]
    z0_ref[...] = jnp.dot(xb, w10_ref[...], preferred_element_type=jnp.float32)
    zs1_ref[...] = (
        jnp.dot(xb, w11_ref[...], preferred_element_type=jnp.float32)
        * dis_ref[...]
    )


@functools.lru_cache(maxsize=None)
def _tc_d1(din, dh):
    return pl.pallas_call(
        _tc_d1_body,
        grid=(_NB,),
        in_specs=[
            pl.BlockSpec((_BN, din), lambda i: (i, 0)),
            pl.BlockSpec((din, dh), lambda i: (0, 0)),
            pl.BlockSpec((din, dh), lambda i: (0, 0)),
            pl.BlockSpec((_BN, 1), lambda i: (i, 0)),
        ],
        out_specs=[
            pl.BlockSpec((_BN, dh), lambda i: (i, 0)),
            pl.BlockSpec((_BN, dh), lambda i: (i, 0)),
        ],
        out_shape=[
            jax.ShapeDtypeStruct((_N, dh), jnp.float32),
            jax.ShapeDtypeStruct((_N, dh), jnp.float32),
        ],
    )


def _tc_d2_body(z0_ref, t1_ref, dis_ref, b1_ref, w20_ref, w21_ref,
                z20_ref, zs2_ref):
    t = t1_ref[...]
    dis = dis_ref[...]
    h = jnp.maximum(z0_ref[...] - dis * (t[0] + t[1]) + b1_ref[...], 0.0)
    z20_ref[...] = jnp.dot(h, w20_ref[...], preferred_element_type=jnp.float32)
    zs2_ref[...] = (
        jnp.dot(h, w21_ref[...], preferred_element_type=jnp.float32) * dis
    )


@functools.lru_cache(maxsize=None)
def _tc_d2(dh, do):
    return pl.pallas_call(
        _tc_d2_body,
        grid=(_NB,),
        in_specs=[
            pl.BlockSpec((_BN, dh), lambda i: (i, 0)),
            pl.BlockSpec((2, _BN, dh), lambda i: (0, i, 0)),
            pl.BlockSpec((_BN, 1), lambda i: (i, 0)),
            pl.BlockSpec((1, dh), lambda i: (0, 0)),
            pl.BlockSpec((dh, do), lambda i: (0, 0)),
            pl.BlockSpec((dh, do), lambda i: (0, 0)),
        ],
        out_specs=[
            pl.BlockSpec((_BN, do), lambda i: (i, 0)),
            pl.BlockSpec((_BN, do), lambda i: (i, 0)),
        ],
        out_shape=[
            jax.ShapeDtypeStruct((_N, do), jnp.float32),
            jax.ShapeDtypeStruct((_N, do), jnp.float32),
        ],
    )


def _tc_fin_body(z20_ref, t2_ref, dis_ref, b2_ref, out_ref):
    t = t2_ref[...]
    o = z20_ref[...] - dis_ref[...] * (t[0] + t[1]) + b2_ref[...]
    m = jnp.max(o, axis=1, keepdims=True)
    lse = jnp.log(jnp.sum(jnp.exp(o - m), axis=1, keepdims=True))
    out_ref[...] = o - m - lse


@functools.lru_cache(maxsize=None)
def _tc_fin(do):
    return pl.pallas_call(
        _tc_fin_body,
        grid=(_NB,),
        in_specs=[
            pl.BlockSpec((_BN, do), lambda i: (i, 0)),
            pl.BlockSpec((2, _BN, do), lambda i: (0, i, 0)),
            pl.BlockSpec((_BN, 1), lambda i: (i, 0)),
            pl.BlockSpec((1, do), lambda i: (0, 0)),
        ],
        out_specs=pl.BlockSpec((_BN, do), lambda i: (i, 0)),
        out_shape=jax.ShapeDtypeStruct((_N, do), jnp.float32),
    )


def kernel(x, edge_index, W10, W11, b1, W20, W21, b2):
    din = x.shape[1]
    dh = W10.shape[1]
    do = W20.shape[1]

    rowpad, colpad, hist = _sc_pre()(edge_index.reshape(2 * _E))
    dis_row = _tc_dis()(hist.reshape(_NW, _HISTR * _HISTW))
    dis = dis_row.reshape(_HISTR * _HISTW, 1)

    z0, zs1 = _tc_d1(din, dh)(x, W10, W11, dis)
    row2d = rowpad.reshape(_NW * _NCH, 128)
    col2d = colpad.reshape(_NW * _NCH, 128)
    t1 = _sc_edge(dh)(zs1, row2d, col2d)
    z20, zs2 = _tc_d2(dh, do)(z0, t1, dis, b1.reshape(1, dh), W20, W21)
    t2 = _sc_edge(do)(zs2, row2d, col2d)
    return _tc_fin(do)(z20, t2, dis, b2.reshape(1, do))


# speedup vs baseline: 17.5095x; 1.0793x over previous
import functools

import jax
import jax.numpy as jnp
from jax import lax
from jax.experimental import pallas as pl
from jax.experimental.pallas import tpu as pltpu
from jax.experimental.pallas import tpu_sc as plsc

_N = 10000
_E = 160000
_NW = 32
_EP = _E // _NW
_EPP = 5120
_NCH = _EPP // 128
_NP = 10240
_RPT = _NP // 16
_DUMMY = _N
_HISTW = 16
_HISTR = 640


def _mesh():
    return plsc.VectorSubcoreMesh(core_axis_name="c", subcore_axis_name="s")


@functools.lru_cache(maxsize=None)
def _sc_pre():
    def body(ei, rowpad, colpad, hist, row_v, col_v, hist_v):
        c = lax.axis_index("c")
        s = lax.axis_index("s")
        wid = c * 16 + s
        base = wid * _EP
        iota16 = lax.iota(jnp.int32, 16)
        zi = iota16 * 0
        zf = zi.astype(jnp.float32)
        ones = zf + 1.0

        def zh(i, carry):
            hist_v[pl.ds(i * 16, 16)] = zf
            return carry
        lax.fori_loop(0, _HISTR, zh, 0)

        pltpu.sync_copy(ei.at[pl.ds(base, _EP)], row_v.at[pl.ds(0, _EP)])
        pltpu.sync_copy(ei.at[pl.ds(_E + base, _EP)], col_v.at[pl.ds(0, _EP)])
        for k in range(7):
            row_v[pl.ds(_EP + 16 * k, 16)] = zi
            col_v[pl.ds(_EP + 16 * k, 16)] = zi
        row_v[pl.ds(_EPP - 16, 16)] = zi
        col_v[pl.ds(_EPP - 16, 16)] = zi

        def eb(j, carry):
            off = j * 16
            r = row_v[pl.ds(off, 16)]
            cc = col_v[pl.ds(off, 16)]
            self_ = r == cc
            col_v[pl.ds(off, 16)] = jnp.where(self_, _DUMMY, cc)
            di = jnp.where(self_, _HISTR * _HISTW - 1, r)
            plsc.addupdate_scatter(hist_v, [di], ones)
            return carry
        lax.fori_loop(0, _EPP // 16, eb, 0)

        pltpu.sync_copy(row_v, rowpad.at[pl.ds(wid * _EPP, _EPP)])
        pltpu.sync_copy(col_v, colpad.at[pl.ds(wid * _EPP, _EPP)])
        pltpu.sync_copy(hist_v, hist.at[wid])

    return pl.kernel(
        body,
        out_type=(
            jax.ShapeDtypeStruct((_NW * _EPP,), jnp.int32),
            jax.ShapeDtypeStruct((_NW * _EPP,), jnp.int32),
            jax.ShapeDtypeStruct((_NW, _HISTR * _HISTW), jnp.float32),
        ),
        mesh=_mesh(),
        scratch_types=[
            pltpu.VMEM((_EPP,), jnp.int32),
            pltpu.VMEM((_EPP,), jnp.int32),
            pltpu.VMEM((_HISTR * _HISTW,), jnp.float32),
        ],
        compiler_params=pltpu.CompilerParams(needs_layout_passes=False),
    )


@functools.lru_cache(maxsize=None)
def _sc_edge(D):
    def body(z, rowi, coli, t_out, ri_v, ci_v, g0, g1, zb_v, t_sh, sem0, sem1):
        c = lax.axis_index("c")
        s = lax.axis_index("s")
        wid = c * 16 + s
        zf = lax.iota(jnp.int32, 16).astype(jnp.float32) * 0.0

        def zb(i, carry):
            for k in range(D // 16):
                zb_v[i, pl.ds(16 * k, 16)] = zf
            return carry
        lax.fori_loop(0, _RPT, zb, 0)
        pltpu.sync_copy(zb_v, t_sh.at[pl.ds(s * _RPT, _RPT)])

        pltpu.sync_copy(rowi.at[pl.ds(wid * _NCH, _NCH)], ri_v)
        pltpu.sync_copy(coli.at[pl.ds(wid * _NCH, _NCH)], ci_v)
        plsc.subcore_barrier()

        def start_g(j, buf, sem):
            pltpu.async_copy(z.at[ri_v.at[j]], buf, sem)

        def wait_g(buf, sem):
            pltpu.make_async_copy(z.at[ri_v.at[0]], buf, sem).wait()

        start_g(0, g0, sem0)

        def lb(k, carry):
            j0 = 2 * k
            j1 = j0 + 1
            wait_g(g0, sem0)
            start_g(j1, g1, sem1)
            pltpu.sync_copy(g0, t_sh.at[ci_v.at[j0]], add=True)
            wait_g(g1, sem1)

            @pl.when(j1 + 1 < _NCH)
            def _():
                start_g(j1 + 1, g0, sem0)

            pltpu.sync_copy(g1, t_sh.at[ci_v.at[j1]], add=True)
            return carry
        lax.fori_loop(0, _NCH // 2, lb, 0)

        plsc.subcore_barrier()
        pltpu.sync_copy(t_sh.at[pl.ds(s * _RPT, _RPT)], zb_v)
        pltpu.sync_copy(zb_v, t_out.at[c, pl.ds(s * _RPT, _RPT)])

    return pl.kernel(
        body,
        out_type=jax.ShapeDtypeStruct((2, _NP, D), jnp.float32),
        mesh=_mesh(),
        scratch_types=[
            pltpu.VMEM((_NCH, 128), jnp.int32),
            pltpu.VMEM((_NCH, 128), jnp.int32),
            pltpu.VMEM((128, D), jnp.float32),
            pltpu.VMEM((128, D), jnp.float32),
            pltpu.VMEM((_RPT, D), jnp.float32),
            pltpu.VMEM_SHARED((_NP, D), jnp.float32),
            pltpu.SemaphoreType.DMA,
            pltpu.SemaphoreType.DMA,
        ],
        compiler_params=pltpu.CompilerParams(
            needs_layout_passes=False, use_tc_tiling_on_sc=False
        ),
    )


def _tc_scale_body(hist_ref, z1_ref, dis_ref, zs1_ref):
    deg = jnp.sum(hist_ref[...], axis=0, keepdims=True)
    safe = jnp.where(deg > 0, deg, 1.0)
    dis = jnp.where(deg > 0, lax.rsqrt(safe), 0.0)
    dis_col = dis.reshape(dis.shape[1], 1)
    dis_ref[...] = dis_col
    zs1_ref[...] = z1_ref[...] * dis_col


_BN = 1024
_NB = 10


@functools.lru_cache(maxsize=None)
def _tc_scale(dh):
    return pl.pallas_call(
        _tc_scale_body,
        grid=(_NB,),
        in_specs=[
            pl.BlockSpec((_NW, _BN), lambda i: (0, i)),
            pl.BlockSpec((_BN, dh), lambda i: (i, 0)),
        ],
        out_specs=[
            pl.BlockSpec((_BN, 1), lambda i: (i, 0)),
            pl.BlockSpec((_BN, dh), lambda i: (i, 0)),
        ],
        out_shape=[
            jax.ShapeDtypeStruct((_NP, 1), jnp.float32),
            jax.ShapeDtypeStruct((_N, dh), jnp.float32),
        ],
    )


def _tc_mm_body(x_ref, w10_ref, w11_ref, z0_ref, z1_ref):
    xb = x_ref[...]
    z0_ref[...] = jnp.dot(xb, w10_ref[...], preferred_element_type=jnp.float32)
    z1_ref[...] = jnp.dot(xb, w11_ref[...], preferred_element_type=jnp.float32)


@functools.lru_cache(maxsize=None)
def _tc_mm(din, dh):
    return pl.pallas_call(
        _tc_mm_body,
        grid=(_NB,),
        in_specs=[
            pl.BlockSpec((_BN, din), lambda i: (i, 0)),
            pl.BlockSpec((din, dh), lambda i: (0, 0)),
            pl.BlockSpec((din, dh), lambda i: (0, 0)),
        ],
        out_specs=[
            pl.BlockSpec((_BN, dh), lambda i: (i, 0)),
            pl.BlockSpec((_BN, dh), lambda i: (i, 0)),
        ],
        out_shape=[
            jax.ShapeDtypeStruct((_N, dh), jnp.float32),
            jax.ShapeDtypeStruct((_N, dh), jnp.float32),
        ],
    )


def _tc_d2_body(z0_ref, t1_ref, dis_ref, b1_ref, w20_ref, w21_ref,
                z20_ref, zs2_ref):
    t = t1_ref[...]
    dis = dis_ref[...]
    h = jnp.maximum(z0_ref[...] - dis * (t[0] + t[1]) + b1_ref[...], 0.0)
    z20_ref[...] = jnp.dot(h, w20_ref[...], preferred_element_type=jnp.float32)
    zs2_ref[...] = (
        jnp.dot(h, w21_ref[...], preferred_element_type=jnp.float32) * dis
    )


@functools.lru_cache(maxsize=None)
def _tc_d2(dh, do):
    return pl.pallas_call(
        _tc_d2_body,
        grid=(_NB,),
        in_specs=[
            pl.BlockSpec((_BN, dh), lambda i: (i, 0)),
            pl.BlockSpec((2, _BN, dh), lambda i: (0, i, 0)),
            pl.BlockSpec((_BN, 1), lambda i: (i, 0)),
            pl.BlockSpec((1, dh), lambda i: (0, 0)),
            pl.BlockSpec((dh, do), lambda i: (0, 0)),
            pl.BlockSpec((dh, do), lambda i: (0, 0)),
        ],
        out_specs=[
            pl.BlockSpec((_BN, do), lambda i: (i, 0)),
            pl.BlockSpec((_BN, do), lambda i: (i, 0)),
        ],
        out_shape=[
            jax.ShapeDtypeStruct((_N, do), jnp.float32),
            jax.ShapeDtypeStruct((_N, do), jnp.float32),
        ],
    )


_BNF = 400
_NBF = _N // _BNF


def _tc_fin_body(z20_ref, t2_ref, dis_ref, b2_ref, out_ref):
    t = t2_ref[...]
    o = z20_ref[...] - dis_ref[...] * (t[0] + t[1]) + b2_ref[...]
    m = jnp.max(o, axis=1, keepdims=True)
    lse = jnp.log(jnp.sum(jnp.exp(o - m), axis=1, keepdims=True))
    out_ref[...] = o - m - lse


@functools.lru_cache(maxsize=None)
def _tc_fin(do):
    return pl.pallas_call(
        _tc_fin_body,
        grid=(_NBF,),
        in_specs=[
            pl.BlockSpec((_BNF, do), lambda i: (i, 0)),
            pl.BlockSpec((2, _BNF, do), lambda i: (0, i, 0)),
            pl.BlockSpec((_BNF, 1), lambda i: (i, 0)),
            pl.BlockSpec((1, do), lambda i: (0, 0)),
        ],
        out_specs=pl.BlockSpec((_BNF, do), lambda i: (i, 0)),
        out_shape=jax.ShapeDtypeStruct((_N, do), jnp.float32),
    )


def kernel(x, edge_index, W10, W11, b1, W20, W21, b2):
    din = x.shape[1]
    dh = W10.shape[1]
    do = W20.shape[1]

    rowpad, colpad, hist = _sc_pre()(edge_index.reshape(2 * _E))
    z0, z1 = _tc_mm(din, dh)(x, W10, W11)
    dis, zs1 = _tc_scale(dh)(hist, z1)

    row2d = rowpad.reshape(_NW * _NCH, 128)
    col2d = colpad.reshape(_NW * _NCH, 128)
    t1 = _sc_edge(dh)(zs1, row2d, col2d)
    z20, zs2 = _tc_d2(dh, do)(z0, t1, dis, b1.reshape(1, dh), W20, W21)
    t2 = _sc_edge(do)(zs2, row2d, col2d)
    return _tc_fin(do)(z20, t2, dis, b2.reshape(1, do))


# speedup vs baseline: 19.3851x; 1.1071x over previous
import functools

import jax
import jax.numpy as jnp
from jax import lax
from jax.experimental import pallas as pl
from jax.experimental.pallas import tpu as pltpu
from jax.experimental.pallas import tpu_sc as plsc

_N = 10000
_E = 160000
_NW = 32
_EP = _E // _NW
_EPP = 5120
_NCH = _EPP // 128
_NP = 10240
_RPT = _NP // 16
_DUMMY = _N
_HISTW = 16
_HISTR = 640


def _mesh():
    return plsc.VectorSubcoreMesh(core_axis_name="c", subcore_axis_name="s")


@functools.lru_cache(maxsize=None)
def _sc_pre():
    def body(ei, rowpad, colpad, hist, row_v, col_v, hist_v):
        c = lax.axis_index("c")
        s = lax.axis_index("s")
        wid = c * 16 + s
        base = wid * _EP
        iota16 = lax.iota(jnp.int32, 16)
        zi = iota16 * 0
        zf = zi.astype(jnp.float32)
        ones = zf + 1.0

        def zh(i, carry):
            hist_v[pl.ds(i * 16, 16)] = zf
            return carry
        lax.fori_loop(0, _HISTR, zh, 0)

        pltpu.sync_copy(ei.at[pl.ds(base, _EP)], row_v.at[pl.ds(0, _EP)])
        pltpu.sync_copy(ei.at[pl.ds(_E + base, _EP)], col_v.at[pl.ds(0, _EP)])
        for k in range(7):
            row_v[pl.ds(_EP + 16 * k, 16)] = zi
            col_v[pl.ds(_EP + 16 * k, 16)] = zi
        row_v[pl.ds(_EPP - 16, 16)] = zi
        col_v[pl.ds(_EPP - 16, 16)] = zi

        def eb(j, carry):
            off = j * 16
            r = row_v[pl.ds(off, 16)]
            cc = col_v[pl.ds(off, 16)]
            self_ = r == cc
            col_v[pl.ds(off, 16)] = jnp.where(self_, _DUMMY, cc)
            di = jnp.where(self_, _HISTR * _HISTW - 1, r)
            plsc.addupdate_scatter(hist_v, [di], ones)
            return carry
        lax.fori_loop(0, _EPP // 16, eb, 0)

        pltpu.sync_copy(row_v, rowpad.at[pl.ds(wid * _EPP, _EPP)])
        pltpu.sync_copy(col_v, colpad.at[pl.ds(wid * _EPP, _EPP)])
        pltpu.sync_copy(hist_v, hist.at[wid])

    return pl.kernel(
        body,
        out_type=(
            jax.ShapeDtypeStruct((_NW * _EPP,), jnp.int32),
            jax.ShapeDtypeStruct((_NW * _EPP,), jnp.int32),
            jax.ShapeDtypeStruct((_NW, _HISTR * _HISTW), jnp.float32),
        ),
        mesh=_mesh(),
        scratch_types=[
            pltpu.VMEM((_EPP,), jnp.int32),
            pltpu.VMEM((_EPP,), jnp.int32),
            pltpu.VMEM((_HISTR * _HISTW,), jnp.float32),
        ],
        compiler_params=pltpu.CompilerParams(needs_layout_passes=False),
    )


@functools.lru_cache(maxsize=None)
def _sc_edge(D):
    def body(z, rowi, coli, t_out, ri_v, ci_v, g0, g1, zb_v, t_sh, sem0, sem1):
        c = lax.axis_index("c")
        s = lax.axis_index("s")
        wid = c * 16 + s
        zf = lax.iota(jnp.int32, 16).astype(jnp.float32) * 0.0

        def zb(i, carry):
            for k in range(D // 16):
                zb_v[i, pl.ds(16 * k, 16)] = zf
            return carry
        lax.fori_loop(0, _RPT, zb, 0)
        pltpu.sync_copy(zb_v, t_sh.at[pl.ds(s * _RPT, _RPT)])

        pltpu.sync_copy(rowi.at[pl.ds(wid * _NCH, _NCH)], ri_v)
        pltpu.sync_copy(coli.at[pl.ds(wid * _NCH, _NCH)], ci_v)
        plsc.subcore_barrier()

        def start_g(j, buf, sem):
            pltpu.async_copy(z.at[ri_v.at[j]], buf, sem)

        def wait_g(buf, sem):
            pltpu.make_async_copy(z.at[ri_v.at[0]], buf, sem).wait()

        start_g(0, g0, sem0)

        def lb(k, carry):
            j0 = 2 * k
            j1 = j0 + 1
            wait_g(g0, sem0)
            start_g(j1, g1, sem1)
            pltpu.sync_copy(g0, t_sh.at[ci_v.at[j0]], add=True)
            wait_g(g1, sem1)

            @pl.when(j1 + 1 < _NCH)
            def _():
                start_g(j1 + 1, g0, sem0)

            pltpu.sync_copy(g1, t_sh.at[ci_v.at[j1]], add=True)
            return carry
        lax.fori_loop(0, _NCH // 2, lb, 0)

        plsc.subcore_barrier()
        pltpu.sync_copy(t_sh.at[pl.ds(s * _RPT, _RPT)], zb_v)
        pltpu.sync_copy(zb_v, t_out.at[c, pl.ds(s * _RPT, _RPT), pl.ds(0, D)])

    return pl.kernel(
        body,
        out_type=jax.ShapeDtypeStruct((2, _NP, 128), jnp.float32),
        mesh=_mesh(),
        scratch_types=[
            pltpu.VMEM((_NCH, 128), jnp.int32),
            pltpu.VMEM((_NCH, 128), jnp.int32),
            pltpu.VMEM((128, D), jnp.float32),
            pltpu.VMEM((128, D), jnp.float32),
            pltpu.VMEM((_RPT, D), jnp.float32),
            pltpu.VMEM_SHARED((_NP, D), jnp.float32),
            pltpu.SemaphoreType.DMA,
            pltpu.SemaphoreType.DMA,
        ],
        compiler_params=pltpu.CompilerParams(
            needs_layout_passes=False, use_tc_tiling_on_sc=False
        ),
    )


def _tc_scale_body(hist_ref, z1_ref, dis_ref, zs1_ref):
    deg = jnp.sum(hist_ref[...], axis=0, keepdims=True)
    safe = jnp.where(deg > 0, deg, 1.0)
    dis = jnp.where(deg > 0, lax.rsqrt(safe), 0.0)
    dis_col = dis.reshape(dis.shape[1], 1)
    dis_ref[...] = dis_col
    zs1_ref[...] = z1_ref[...] * dis_col


_BN = 1024
_NB = 10


@functools.lru_cache(maxsize=None)
def _tc_scale(dh):
    return pl.pallas_call(
        _tc_scale_body,
        grid=(_NB,),
        in_specs=[
            pl.BlockSpec((_NW, _BN), lambda i: (0, i)),
            pl.BlockSpec((_BN, dh), lambda i: (i, 0)),
        ],
        out_specs=[
            pl.BlockSpec((_BN, 1), lambda i: (i, 0)),
            pl.BlockSpec((_BN, dh), lambda i: (i, 0)),
        ],
        out_shape=[
            jax.ShapeDtypeStruct((_NP, 1), jnp.float32),
            jax.ShapeDtypeStruct((_N, dh), jnp.float32),
        ],
    )


def _tc_mm_body(x_ref, w10_ref, w11_ref, z0_ref, z1_ref):
    xb = x_ref[...]
    z0_ref[...] = jnp.dot(xb, w10_ref[...], preferred_element_type=jnp.float32)
    z1_ref[...] = jnp.dot(xb, w11_ref[...], preferred_element_type=jnp.float32)


@functools.lru_cache(maxsize=None)
def _tc_mm(din, dh):
    return pl.pallas_call(
        _tc_mm_body,
        grid=(_NB,),
        in_specs=[
            pl.BlockSpec((_BN, din), lambda i: (i, 0)),
            pl.BlockSpec((din, dh), lambda i: (0, 0)),
            pl.BlockSpec((din, dh), lambda i: (0, 0)),
        ],
        out_specs=[
            pl.BlockSpec((_BN, dh), lambda i: (i, 0)),
            pl.BlockSpec((_BN, dh), lambda i: (i, 0)),
        ],
        out_shape=[
            jax.ShapeDtypeStruct((_N, dh), jnp.float32),
            jax.ShapeDtypeStruct((_N, dh), jnp.float32),
        ],
    )


def _tc_d2_body(z0_ref, t1_ref, dis_ref, b1_ref, w20_ref, w21_ref,
                z20_ref, zs2_ref):
    dh = z0_ref.shape[1]
    t = t1_ref[0, :, :dh] + t1_ref[1, :, :dh]
    dis = dis_ref[...]
    h = jnp.maximum(z0_ref[...] - dis * t + b1_ref[...], 0.0)
    z20_ref[...] = jnp.dot(h, w20_ref[...], preferred_element_type=jnp.float32)
    zs2_ref[...] = (
        jnp.dot(h, w21_ref[...], preferred_element_type=jnp.float32) * dis
    )


@functools.lru_cache(maxsize=None)
def _tc_d2(dh, do):
    return pl.pallas_call(
        _tc_d2_body,
        grid=(_NB,),
        in_specs=[
            pl.BlockSpec((_BN, dh), lambda i: (i, 0)),
            pl.BlockSpec((2, _BN, 128), lambda i: (0, i, 0)),
            pl.BlockSpec((_BN, 1), lambda i: (i, 0)),
            pl.BlockSpec((1, dh), lambda i: (0, 0)),
            pl.BlockSpec((dh, do), lambda i: (0, 0)),
            pl.BlockSpec((dh, do), lambda i: (0, 0)),
        ],
        out_specs=[
            pl.BlockSpec((_BN, do), lambda i: (i, 0)),
            pl.BlockSpec((_BN, do), lambda i: (i, 0)),
        ],
        out_shape=[
            jax.ShapeDtypeStruct((_N, do), jnp.float32),
            jax.ShapeDtypeStruct((_N, do), jnp.float32),
        ],
    )


_BNF = 1024
_NBF = 10


def _tc_fin_body(z20_ref, t2_ref, dis_ref, b2_ref, out_ref):
    do = z20_ref.shape[1]
    t = t2_ref[0, :, :do] + t2_ref[1, :, :do]
    o = z20_ref[...] - dis_ref[...] * t + b2_ref[...]
    m = jnp.max(o, axis=1, keepdims=True)
    lse = jnp.log(jnp.sum(jnp.exp(o - m), axis=1, keepdims=True))
    out_ref[...] = o - m - lse


@functools.lru_cache(maxsize=None)
def _tc_fin(do):
    return pl.pallas_call(
        _tc_fin_body,
        grid=(_NBF,),
        in_specs=[
            pl.BlockSpec((_BNF, do), lambda i: (i, 0)),
            pl.BlockSpec((2, _BNF, 128), lambda i: (0, i, 0)),
            pl.BlockSpec((_BNF, 1), lambda i: (i, 0)),
            pl.BlockSpec((1, do), lambda i: (0, 0)),
        ],
        out_specs=pl.BlockSpec((_BNF, do), lambda i: (i, 0)),
        out_shape=jax.ShapeDtypeStruct((_N, do), jnp.float32),
    )


def kernel(x, edge_index, W10, W11, b1, W20, W21, b2):
    din = x.shape[1]
    dh = W10.shape[1]
    do = W20.shape[1]

    rowpad, colpad, hist = _sc_pre()(edge_index.reshape(2 * _E))
    z0, z1 = _tc_mm(din, dh)(x, W10, W11)
    dis, zs1 = _tc_scale(dh)(hist, z1)

    row2d = rowpad.reshape(_NW * _NCH, 128)
    col2d = colpad.reshape(_NW * _NCH, 128)
    t1 = _sc_edge(dh)(zs1, row2d, col2d)
    z20, zs2 = _tc_d2(dh, do)(z0, t1, dis, b1.reshape(1, dh), W20, W21)
    t2 = _sc_edge(do)(zs2, row2d, col2d)
    return _tc_fin(do)(z20, t2, dis, b2.reshape(1, do))


# speedup vs baseline: 22.4978x; 1.1606x over previous
import functools

import jax
import jax.numpy as jnp
from jax import lax
from jax.experimental import pallas as pl
from jax.experimental.pallas import tpu as pltpu
from jax.experimental.pallas import tpu_sc as plsc

_N = 10000
_E = 160000
_NW = 32
_EP = _E // _NW
_EPP = 5120
_CH = 512
_NCH = _EPP // _CH
_NP = 10240
_RPT = _NP // 16
_DUMMY = _N
_HISTW = 16
_HISTR = 640


def _mesh():
    return plsc.VectorSubcoreMesh(core_axis_name="c", subcore_axis_name="s")


@functools.lru_cache(maxsize=None)
def _sc_pre():
    def body(ei, rowpad, colpad, hist, row_v, col_v, hist_v):
        c = lax.axis_index("c")
        s = lax.axis_index("s")
        wid = c * 16 + s
        base = wid * _EP
        iota16 = lax.iota(jnp.int32, 16)
        zi = iota16 * 0
        zf = zi.astype(jnp.float32)
        ones = zf + 1.0

        def zh(i, carry):
            hist_v[pl.ds(i * 16, 16)] = zf
            return carry
        lax.fori_loop(0, _HISTR, zh, 0)

        pltpu.sync_copy(ei.at[pl.ds(base, _EP)], row_v.at[pl.ds(0, _EP)])
        pltpu.sync_copy(ei.at[pl.ds(_E + base, _EP)], col_v.at[pl.ds(0, _EP)])
        for k in range(7):
            row_v[pl.ds(_EP + 16 * k, 16)] = zi
            col_v[pl.ds(_EP + 16 * k, 16)] = zi
        row_v[pl.ds(_EPP - 16, 16)] = zi
        col_v[pl.ds(_EPP - 16, 16)] = zi

        def eb(j, carry):
            off = j * 16
            r = row_v[pl.ds(off, 16)]
            cc = col_v[pl.ds(off, 16)]
            self_ = r == cc
            col_v[pl.ds(off, 16)] = jnp.where(self_, _DUMMY, cc)
            di = jnp.where(self_, _HISTR * _HISTW - 1, r)
            plsc.addupdate_scatter(hist_v, [di], ones)
            return carry
        lax.fori_loop(0, _EPP // 16, eb, 0)

        pltpu.sync_copy(row_v, rowpad.at[pl.ds(wid * _EPP, _EPP)])
        pltpu.sync_copy(col_v, colpad.at[pl.ds(wid * _EPP, _EPP)])
        pltpu.sync_copy(hist_v, hist.at[wid])

    return pl.kernel(
        body,
        out_type=(
            jax.ShapeDtypeStruct((_NW * _EPP,), jnp.int32),
            jax.ShapeDtypeStruct((_NW * _EPP,), jnp.int32),
            jax.ShapeDtypeStruct((_NW, _HISTR * _HISTW), jnp.float32),
        ),
        mesh=_mesh(),
        scratch_types=[
            pltpu.VMEM((_EPP,), jnp.int32),
            pltpu.VMEM((_EPP,), jnp.int32),
            pltpu.VMEM((_HISTR * _HISTW,), jnp.float32),
        ],
        compiler_params=pltpu.CompilerParams(needs_layout_passes=False),
    )


@functools.lru_cache(maxsize=None)
def _sc_edge(D):
    def body(z, rowi, coli, t_out, ri_v, ci_v, g0, g1, zb_v, t_sh, sem0, sem1):
        c = lax.axis_index("c")
        s = lax.axis_index("s")
        wid = c * 16 + s
        zf = lax.iota(jnp.int32, 16).astype(jnp.float32) * 0.0

        def zb(i, carry):
            for k in range(D // 16):
                zb_v[i, pl.ds(16 * k, 16)] = zf
            return carry
        lax.fori_loop(0, _RPT, zb, 0)
        pltpu.sync_copy(zb_v, t_sh.at[pl.ds(s * _RPT, _RPT)])

        pltpu.sync_copy(rowi.at[pl.ds(wid * _NCH, _NCH)], ri_v)
        pltpu.sync_copy(coli.at[pl.ds(wid * _NCH, _NCH)], ci_v)
        plsc.subcore_barrier()

        def start_g(j, buf, sem):
            pltpu.async_copy(z.at[ri_v.at[j]], buf, sem)

        def wait_g(buf, sem):
            pltpu.make_async_copy(z.at[ri_v.at[0]], buf, sem).wait()

        start_g(0, g0, sem0)

        def lb(k, carry):
            j0 = 2 * k
            j1 = j0 + 1
            wait_g(g0, sem0)
            start_g(j1, g1, sem1)
            pltpu.sync_copy(g0, t_sh.at[ci_v.at[j0]], add=True)
            wait_g(g1, sem1)

            @pl.when(j1 + 1 < _NCH)
            def _():
                start_g(j1 + 1, g0, sem0)

            pltpu.sync_copy(g1, t_sh.at[ci_v.at[j1]], add=True)
            return carry
        lax.fori_loop(0, _NCH // 2, lb, 0)

        plsc.subcore_barrier()
        pltpu.sync_copy(t_sh.at[pl.ds(s * _RPT, _RPT)], zb_v)
        pltpu.sync_copy(zb_v, t_out.at[c, pl.ds(s * _RPT, _RPT), pl.ds(0, D)])

    return pl.kernel(
        body,
        out_type=jax.ShapeDtypeStruct((2, _NP, 128), jnp.float32),
        mesh=_mesh(),
        scratch_types=[
            pltpu.VMEM((_NCH, _CH), jnp.int32),
            pltpu.VMEM((_NCH, _CH), jnp.int32),
            pltpu.VMEM((_CH, D), jnp.float32),
            pltpu.VMEM((_CH, D), jnp.float32),
            pltpu.VMEM((_RPT, D), jnp.float32),
            pltpu.VMEM_SHARED((_NP, D), jnp.float32),
            pltpu.SemaphoreType.DMA,
            pltpu.SemaphoreType.DMA,
        ],
        compiler_params=pltpu.CompilerParams(
            needs_layout_passes=False, use_tc_tiling_on_sc=False
        ),
    )


def _tc_scale_body(hist_ref, z1_ref, dis_ref, zs1_ref):
    deg = jnp.sum(hist_ref[...], axis=0, keepdims=True)
    safe = jnp.where(deg > 0, deg, 1.0)
    dis = jnp.where(deg > 0, lax.rsqrt(safe), 0.0)
    dis_col = dis.reshape(dis.shape[1], 1)
    dis_ref[...] = dis_col
    zs1_ref[...] = z1_ref[...] * dis_col


_BN = 1024
_NB = 10


@functools.lru_cache(maxsize=None)
def _tc_scale(dh):
    return pl.pallas_call(
        _tc_scale_body,
        grid=(_NB,),
        in_specs=[
            pl.BlockSpec((_NW, _BN), lambda i: (0, i)),
            pl.BlockSpec((_BN, dh), lambda i: (i, 0)),
        ],
        out_specs=[
            pl.BlockSpec((_BN, 1), lambda i: (i, 0)),
            pl.BlockSpec((_BN, dh), lambda i: (i, 0)),
        ],
        out_shape=[
            jax.ShapeDtypeStruct((_NP, 1), jnp.float32),
            jax.ShapeDtypeStruct((_N, dh), jnp.float32),
        ],
    )


def _tc_mm_body(x_ref, w10_ref, w11_ref, z0_ref, z1_ref):
    xb = x_ref[...]
    z0_ref[...] = jnp.dot(xb, w10_ref[...], preferred_element_type=jnp.float32)
    z1_ref[...] = jnp.dot(xb, w11_ref[...], preferred_element_type=jnp.float32)


@functools.lru_cache(maxsize=None)
def _tc_mm(din, dh):
    return pl.pallas_call(
        _tc_mm_body,
        grid=(_NB,),
        in_specs=[
            pl.BlockSpec((_BN, din), lambda i: (i, 0)),
            pl.BlockSpec((din, dh), lambda i: (0, 0)),
            pl.BlockSpec((din, dh), lambda i: (0, 0)),
        ],
        out_specs=[
            pl.BlockSpec((_BN, dh), lambda i: (i, 0)),
            pl.BlockSpec((_BN, dh), lambda i: (i, 0)),
        ],
        out_shape=[
            jax.ShapeDtypeStruct((_N, dh), jnp.float32),
            jax.ShapeDtypeStruct((_N, dh), jnp.float32),
        ],
    )


def _tc_d2_body(z0_ref, t1_ref, dis_ref, b1_ref, w20_ref, w21_ref,
                z20_ref, zs2_ref):
    dh = z0_ref.shape[1]
    t = t1_ref[0, :, :dh] + t1_ref[1, :, :dh]
    dis = dis_ref[...]
    h = jnp.maximum(z0_ref[...] - dis * t + b1_ref[...], 0.0)
    z20_ref[...] = jnp.dot(h, w20_ref[...], preferred_element_type=jnp.float32)
    zs2_ref[...] = (
        jnp.dot(h, w21_ref[...], preferred_element_type=jnp.float32) * dis
    )


@functools.lru_cache(maxsize=None)
def _tc_d2(dh, do):
    return pl.pallas_call(
        _tc_d2_body,
        grid=(_NB,),
        in_specs=[
            pl.BlockSpec((_BN, dh), lambda i: (i, 0)),
            pl.BlockSpec((2, _BN, 128), lambda i: (0, i, 0)),
            pl.BlockSpec((_BN, 1), lambda i: (i, 0)),
            pl.BlockSpec((1, dh), lambda i: (0, 0)),
            pl.BlockSpec((dh, do), lambda i: (0, 0)),
            pl.BlockSpec((dh, do), lambda i: (0, 0)),
        ],
        out_specs=[
            pl.BlockSpec((_BN, do), lambda i: (i, 0)),
            pl.BlockSpec((_BN, do), lambda i: (i, 0)),
        ],
        out_shape=[
            jax.ShapeDtypeStruct((_N, do), jnp.float32),
            jax.ShapeDtypeStruct((_N, do), jnp.float32),
        ],
    )


_BNF = 1024
_NBF = 10


def _tc_fin_body(z20_ref, t2_ref, dis_ref, b2_ref, out_ref):
    do = z20_ref.shape[1]
    t = t2_ref[0, :, :do] + t2_ref[1, :, :do]
    o = z20_ref[...] - dis_ref[...] * t + b2_ref[...]
    m = jnp.max(o, axis=1, keepdims=True)
    lse = jnp.log(jnp.sum(jnp.exp(o - m), axis=1, keepdims=True))
    out_ref[...] = o - m - lse


@functools.lru_cache(maxsize=None)
def _tc_fin(do):
    return pl.pallas_call(
        _tc_fin_body,
        grid=(_NBF,),
        in_specs=[
            pl.BlockSpec((_BNF, do), lambda i: (i, 0)),
            pl.BlockSpec((2, _BNF, 128), lambda i: (0, i, 0)),
            pl.BlockSpec((_BNF, 1), lambda i: (i, 0)),
            pl.BlockSpec((1, do), lambda i: (0, 0)),
        ],
        out_specs=pl.BlockSpec((_BNF, do), lambda i: (i, 0)),
        out_shape=jax.ShapeDtypeStruct((_N, do), jnp.float32),
    )


def kernel(x, edge_index, W10, W11, b1, W20, W21, b2):
    din = x.shape[1]
    dh = W10.shape[1]
    do = W20.shape[1]

    rowpad, colpad, hist = _sc_pre()(edge_index.reshape(2 * _E))
    z0, z1 = _tc_mm(din, dh)(x, W10, W11)
    dis, zs1 = _tc_scale(dh)(hist, z1)

    row2d = rowpad.reshape(_NW * _NCH, _CH)
    col2d = colpad.reshape(_NW * _NCH, _CH)
    t1 = _sc_edge(dh)(zs1, row2d, col2d)
    z20, zs2 = _tc_d2(dh, do)(z0, t1, dis, b1.reshape(1, dh), W20, W21)
    t2 = _sc_edge(do)(zs2, row2d, col2d)
    return _tc_fin(do)(z20, t2, dis, b2.reshape(1, do))


# speedup vs baseline: 23.0581x; 1.0249x over previous
import functools

import jax
import jax.numpy as jnp
from jax import lax
from jax.experimental import pallas as pl
from jax.experimental.pallas import tpu as pltpu
from jax.experimental.pallas import tpu_sc as plsc

_N = 10000
_E = 160000
_NW = 32
_EP = _E // _NW
_EPP = 5120
_CH = 1024
_NCH = _EPP // _CH
_NP = 10240
_RPT = _NP // 16
_DUMMY = _N
_HISTW = 16
_HISTR = 640


def _mesh():
    return plsc.VectorSubcoreMesh(core_axis_name="c", subcore_axis_name="s")


@functools.lru_cache(maxsize=None)
def _sc_pre():
    def body(ei, rowpad, colpad, hist, row_v, col_v, hist_v):
        c = lax.axis_index("c")
        s = lax.axis_index("s")
        wid = c * 16 + s
        base = wid * _EP
        iota16 = lax.iota(jnp.int32, 16)
        zi = iota16 * 0
        zf = zi.astype(jnp.float32)
        ones = zf + 1.0

        def zh(i, carry):
            hist_v[pl.ds(i * 16, 16)] = zf
            return carry
        lax.fori_loop(0, _HISTR, zh, 0)

        pltpu.sync_copy(ei.at[pl.ds(base, _EP)], row_v.at[pl.ds(0, _EP)])
        pltpu.sync_copy(ei.at[pl.ds(_E + base, _EP)], col_v.at[pl.ds(0, _EP)])
        for k in range(7):
            row_v[pl.ds(_EP + 16 * k, 16)] = zi
            col_v[pl.ds(_EP + 16 * k, 16)] = zi
        row_v[pl.ds(_EPP - 16, 16)] = zi
        col_v[pl.ds(_EPP - 16, 16)] = zi

        def eb(j, carry):
            off = j * 16
            r = row_v[pl.ds(off, 16)]
            cc = col_v[pl.ds(off, 16)]
            self_ = r == cc
            col_v[pl.ds(off, 16)] = jnp.where(self_, _DUMMY, cc)
            di = jnp.where(self_, _HISTR * _HISTW - 1, r)
            plsc.addupdate_scatter(hist_v, [di], ones)
            return carry
        lax.fori_loop(0, _EPP // 16, eb, 0)

        pltpu.sync_copy(row_v, rowpad.at[pl.ds(wid * _EPP, _EPP)])
        pltpu.sync_copy(col_v, colpad.at[pl.ds(wid * _EPP, _EPP)])
        pltpu.sync_copy(hist_v, hist.at[wid])

    return pl.kernel(
        body,
        out_type=(
            jax.ShapeDtypeStruct((_NW * _EPP,), jnp.int32),
            jax.ShapeDtypeStruct((_NW * _EPP,), jnp.int32),
            jax.ShapeDtypeStruct((_NW, _HISTR * _HISTW), jnp.float32),
        ),
        mesh=_mesh(),
        scratch_types=[
            pltpu.VMEM((_EPP,), jnp.int32),
            pltpu.VMEM((_EPP,), jnp.int32),
            pltpu.VMEM((_HISTR * _HISTW,), jnp.float32),
        ],
        compiler_params=pltpu.CompilerParams(needs_layout_passes=False),
    )


@functools.lru_cache(maxsize=None)
def _sc_edge(D):
    def body(z, rowi, coli, t_out, ri_v, ci_v, g0, g1, zb_v, t_sh, sem0, sem1):
        c = lax.axis_index("c")
        s = lax.axis_index("s")
        wid = c * 16 + s
        zf = lax.iota(jnp.int32, 16).astype(jnp.float32) * 0.0

        def zb(i, carry):
            for k in range(D // 16):
                zb_v[i, pl.ds(16 * k, 16)] = zf
            return carry
        lax.fori_loop(0, _RPT, zb, 0)
        pltpu.sync_copy(zb_v, t_sh.at[pl.ds(s * _RPT, _RPT)])

        pltpu.sync_copy(rowi.at[pl.ds(wid * _NCH, _NCH)], ri_v)
        pltpu.sync_copy(coli.at[pl.ds(wid * _NCH, _NCH)], ci_v)
        plsc.subcore_barrier()

        def start_g(j, buf, sem):
            pltpu.async_copy(z.at[ri_v.at[j]], buf, sem)

        def wait_g(buf, sem):
            pltpu.make_async_copy(z.at[ri_v.at[0]], buf, sem).wait()

        bufs = [(g0, sem0), (g1, sem1)]
        start_g(0, g0, sem0)
        for j in range(_NCH):
            buf, sem = bufs[j % 2]
            wait_g(buf, sem)
            if j + 1 < _NCH:
                start_g(j + 1, *bufs[(j + 1) % 2])
            pltpu.sync_copy(buf, t_sh.at[ci_v.at[j]], add=True)

        plsc.subcore_barrier()
        pltpu.sync_copy(t_sh.at[pl.ds(s * _RPT, _RPT)], zb_v)
        pltpu.sync_copy(zb_v, t_out.at[c, pl.ds(s * _RPT, _RPT), pl.ds(0, D)])

    return pl.kernel(
        body,
        out_type=jax.ShapeDtypeStruct((2, _NP, 128), jnp.float32),
        mesh=_mesh(),
        scratch_types=[
            pltpu.VMEM((_NCH, _CH), jnp.int32),
            pltpu.VMEM((_NCH, _CH), jnp.int32),
            pltpu.VMEM((_CH, D), jnp.float32),
            pltpu.VMEM((_CH, D), jnp.float32),
            pltpu.VMEM((_RPT, D), jnp.float32),
            pltpu.VMEM_SHARED((_NP, D), jnp.float32),
            pltpu.SemaphoreType.DMA,
            pltpu.SemaphoreType.DMA,
        ],
        compiler_params=pltpu.CompilerParams(
            needs_layout_passes=False, use_tc_tiling_on_sc=False
        ),
    )


def _tc_scale_body(hist_ref, z1_ref, dis_ref, zs1_ref):
    deg = jnp.sum(hist_ref[...], axis=0, keepdims=True)
    safe = jnp.where(deg > 0, deg, 1.0)
    dis = jnp.where(deg > 0, lax.rsqrt(safe), 0.0)
    dis_col = dis.reshape(dis.shape[1], 1)
    dis_ref[...] = dis_col
    zs1_ref[...] = z1_ref[...] * dis_col


_BN = 1024
_NB = 10


@functools.lru_cache(maxsize=None)
def _tc_scale(dh):
    return pl.pallas_call(
        _tc_scale_body,
        grid=(_NB,),
        in_specs=[
            pl.BlockSpec((_NW, _BN), lambda i: (0, i)),
            pl.BlockSpec((_BN, dh), lambda i: (i, 0)),
        ],
        out_specs=[
            pl.BlockSpec((_BN, 1), lambda i: (i, 0)),
            pl.BlockSpec((_BN, dh), lambda i: (i, 0)),
        ],
        out_shape=[
            jax.ShapeDtypeStruct((_NP, 1), jnp.float32),
            jax.ShapeDtypeStruct((_N, dh), jnp.float32),
        ],
    )


def _tc_mm_body(x_ref, w10_ref, w11_ref, z0_ref, z1_ref):
    xb = x_ref[...]
    z0_ref[...] = jnp.dot(xb, w10_ref[...], preferred_element_type=jnp.float32)
    z1_ref[...] = jnp.dot(xb, w11_ref[...], preferred_element_type=jnp.float32)


@functools.lru_cache(maxsize=None)
def _tc_mm(din, dh):
    return pl.pallas_call(
        _tc_mm_body,
        grid=(_NB,),
        in_specs=[
            pl.BlockSpec((_BN, din), lambda i: (i, 0)),
            pl.BlockSpec((din, dh), lambda i: (0, 0)),
            pl.BlockSpec((din, dh), lambda i: (0, 0)),
        ],
        out_specs=[
            pl.BlockSpec((_BN, dh), lambda i: (i, 0)),
            pl.BlockSpec((_BN, dh), lambda i: (i, 0)),
        ],
        out_shape=[
            jax.ShapeDtypeStruct((_N, dh), jnp.float32),
            jax.ShapeDtypeStruct((_N, dh), jnp.float32),
        ],
    )


def _tc_d2_body(z0_ref, t1_ref, dis_ref, b1_ref, w20_ref, w21_ref,
                z20_ref, zs2_ref):
    dh = z0_ref.shape[1]
    t = t1_ref[0, :, :dh] + t1_ref[1, :, :dh]
    dis = dis_ref[...]
    h = jnp.maximum(z0_ref[...] - dis * t + b1_ref[...], 0.0)
    z20_ref[...] = jnp.dot(h, w20_ref[...], preferred_element_type=jnp.float32)
    zs2_ref[...] = (
        jnp.dot(h, w21_ref[...], preferred_element_type=jnp.float32) * dis
    )


@functools.lru_cache(maxsize=None)
def _tc_d2(dh, do):
    return pl.pallas_call(
        _tc_d2_body,
        grid=(_NB,),
        in_specs=[
            pl.BlockSpec((_BN, dh), lambda i: (i, 0)),
            pl.BlockSpec((2, _BN, 128), lambda i: (0, i, 0)),
            pl.BlockSpec((_BN, 1), lambda i: (i, 0)),
            pl.BlockSpec((1, dh), lambda i: (0, 0)),
            pl.BlockSpec((dh, do), lambda i: (0, 0)),
            pl.BlockSpec((dh, do), lambda i: (0, 0)),
        ],
        out_specs=[
            pl.BlockSpec((_BN, do), lambda i: (i, 0)),
            pl.BlockSpec((_BN, do), lambda i: (i, 0)),
        ],
        out_shape=[
            jax.ShapeDtypeStruct((_N, do), jnp.float32),
            jax.ShapeDtypeStruct((_N, do), jnp.float32),
        ],
    )


_BNF = 1024
_NBF = 10


def _tc_fin_body(z20_ref, t2_ref, dis_ref, b2_ref, out_ref):
    do = z20_ref.shape[1]
    t = t2_ref[0, :, :do] + t2_ref[1, :, :do]
    o = z20_ref[...] - dis_ref[...] * t + b2_ref[...]
    m = jnp.max(o, axis=1, keepdims=True)
    lse = jnp.log(jnp.sum(jnp.exp(o - m), axis=1, keepdims=True))
    out_ref[...] = o - m - lse


@functools.lru_cache(maxsize=None)
def _tc_fin(do):
    return pl.pallas_call(
        _tc_fin_body,
        grid=(_NBF,),
        in_specs=[
            pl.BlockSpec((_BNF, do), lambda i: (i, 0)),
            pl.BlockSpec((2, _BNF, 128), lambda i: (0, i, 0)),
            pl.BlockSpec((_BNF, 1), lambda i: (i, 0)),
            pl.BlockSpec((1, do), lambda i: (0, 0)),
        ],
        out_specs=pl.BlockSpec((_BNF, do), lambda i: (i, 0)),
        out_shape=jax.ShapeDtypeStruct((_N, do), jnp.float32),
    )


def kernel(x, edge_index, W10, W11, b1, W20, W21, b2):
    din = x.shape[1]
    dh = W10.shape[1]
    do = W20.shape[1]

    rowpad, colpad, hist = _sc_pre()(edge_index.reshape(2 * _E))
    z0, z1 = _tc_mm(din, dh)(x, W10, W11)
    dis, zs1 = _tc_scale(dh)(hist, z1)

    row2d = rowpad.reshape(_NW * _NCH, _CH)
    col2d = colpad.reshape(_NW * _NCH, _CH)
    t1 = _sc_edge(dh)(zs1, row2d, col2d)
    z20, zs2 = _tc_d2(dh, do)(z0, t1, dis, b1.reshape(1, dh), W20, W21)
    t2 = _sc_edge(do)(zs2, row2d, col2d)
    return _tc_fin(do)(z20, t2, dis, b2.reshape(1, do))


# speedup vs baseline: 23.3566x; 1.0129x over previous
import functools

import jax
import jax.numpy as jnp
from jax import lax
from jax.experimental import pallas as pl
from jax.experimental.pallas import tpu as pltpu
from jax.experimental.pallas import tpu_sc as plsc

_N = 10000
_E = 160000
_NW = 32
_EP = _E // _NW
_EPP = 5120
_CH = 1024
_NCH = _EPP // _CH
_NP = 10240
_RPT = _NP // 16
_DUMMY = _N
_HISTW = 16
_HISTR = 640


def _mesh():
    return plsc.VectorSubcoreMesh(core_axis_name="c", subcore_axis_name="s")


@functools.lru_cache(maxsize=None)
def _sc_pre():
    def body(ei, rowpad, colpad, hist, row_v, col_v, hist_v):
        c = lax.axis_index("c")
        s = lax.axis_index("s")
        wid = c * 16 + s
        base = wid * _EP
        iota16 = lax.iota(jnp.int32, 16)
        zi = iota16 * 0
        zf = zi.astype(jnp.float32)
        ones = zf + 1.0

        def zh(i, carry):
            hist_v[pl.ds(i * 16, 16)] = zf
            return carry
        lax.fori_loop(0, _HISTR, zh, 0)

        pltpu.sync_copy(ei.at[pl.ds(base, _EP)], row_v.at[pl.ds(0, _EP)])
        pltpu.sync_copy(ei.at[pl.ds(_E + base, _EP)], col_v.at[pl.ds(0, _EP)])
        for k in range(7):
            row_v[pl.ds(_EP + 16 * k, 16)] = zi
            col_v[pl.ds(_EP + 16 * k, 16)] = zi
        row_v[pl.ds(_EPP - 16, 16)] = zi
        col_v[pl.ds(_EPP - 16, 16)] = zi

        def eb(j, carry):
            off = j * 16
            r = row_v[pl.ds(off, 16)]
            cc = col_v[pl.ds(off, 16)]
            self_ = r == cc
            col_v[pl.ds(off, 16)] = jnp.where(self_, _DUMMY, cc)
            di = jnp.where(self_, _HISTR * _HISTW - 1, r)
            plsc.addupdate_scatter(hist_v, [di], ones)
            return carry
        lax.fori_loop(0, _EPP // 16, eb, 0)

        pltpu.sync_copy(row_v, rowpad.at[pl.ds(wid * _EPP, _EPP)])
        pltpu.sync_copy(col_v, colpad.at[pl.ds(wid * _EPP, _EPP)])
        pltpu.sync_copy(hist_v, hist.at[wid])

    return pl.kernel(
        body,
        out_type=(
            jax.ShapeDtypeStruct((_NW * _EPP,), jnp.int32),
            jax.ShapeDtypeStruct((_NW * _EPP,), jnp.int32),
            jax.ShapeDtypeStruct((_NW, _HISTR * _HISTW), jnp.float32),
        ),
        mesh=_mesh(),
        scratch_types=[
            pltpu.VMEM((_EPP,), jnp.int32),
            pltpu.VMEM((_EPP,), jnp.int32),
            pltpu.VMEM((_HISTR * _HISTW,), jnp.float32),
        ],
        compiler_params=pltpu.CompilerParams(needs_layout_passes=False),
    )


@functools.lru_cache(maxsize=None)
def _sc_edge(D):
    def body(z, rowi, coli, t_out, ri_v, ci_v, g0, g1, zb_v, t_sh, sem0, sem1):
        c = lax.axis_index("c")
        s = lax.axis_index("s")
        wid = c * 16 + s
        zf = lax.iota(jnp.int32, 16).astype(jnp.float32) * 0.0

        def zb(i, carry):
            for k in range(D // 16):
                zb_v[i, pl.ds(16 * k, 16)] = zf
            return carry
        lax.fori_loop(0, _RPT, zb, 0)
        pltpu.sync_copy(zb_v, t_sh.at[pl.ds(s * _RPT, _RPT)])

        pltpu.sync_copy(rowi.at[pl.ds(wid * _NCH, _NCH)], ri_v)
        pltpu.sync_copy(coli.at[pl.ds(wid * _NCH, _NCH)], ci_v)
        plsc.subcore_barrier()

        def start_g(j, buf, sem):
            pltpu.async_copy(z.at[ri_v.at[j]], buf, sem)

        def wait_g(buf, sem):
            pltpu.make_async_copy(z.at[ri_v.at[0]], buf, sem).wait()

        bufs = [(g0, sem0), (g1, sem1)]
        start_g(0, g0, sem0)
        for j in range(_NCH):
            buf, sem = bufs[j % 2]
            wait_g(buf, sem)
            if j + 1 < _NCH:
                start_g(j + 1, *bufs[(j + 1) % 2])
            pltpu.sync_copy(buf, t_sh.at[ci_v.at[j]], add=True)

        plsc.subcore_barrier()
        pltpu.sync_copy(t_sh.at[pl.ds(s * _RPT, _RPT)], zb_v)
        pltpu.sync_copy(zb_v, t_out.at[c, pl.ds(s * _RPT, _RPT), pl.ds(0, D)])

    return pl.kernel(
        body,
        out_type=jax.ShapeDtypeStruct((2, _NP, 128), jnp.float32),
        mesh=_mesh(),
        scratch_types=[
            pltpu.VMEM((_NCH, _CH), jnp.int32),
            pltpu.VMEM((_NCH, _CH), jnp.int32),
            pltpu.VMEM((_CH, D), jnp.float32),
            pltpu.VMEM((_CH, D), jnp.float32),
            pltpu.VMEM((_RPT, D), jnp.float32),
            pltpu.VMEM_SHARED((_NP, D), jnp.float32),
            pltpu.SemaphoreType.DMA,
            pltpu.SemaphoreType.DMA,
        ],
        compiler_params=pltpu.CompilerParams(
            needs_layout_passes=False, use_tc_tiling_on_sc=False
        ),
    )


_BN = 1024
_NB = 10


def _tc_d1_body(x_ref, w10_ref, w11_ref, hist_ref, z0_ref, dis_ref, zs1_ref):
    deg = jnp.sum(hist_ref[...], axis=0, keepdims=True)
    safe = jnp.where(deg > 0, deg, 1.0)
    dis = jnp.where(deg > 0, lax.rsqrt(safe), 0.0)
    dis_col = dis.reshape(dis.shape[1], 1)
    dis_ref[...] = dis_col
    xb = x_ref[...]
    z0_ref[...] = jnp.dot(xb, w10_ref[...], preferred_element_type=jnp.float32)
    zs1_ref[...] = (
        jnp.dot(xb, w11_ref[...], preferred_element_type=jnp.float32) * dis_col
    )


@functools.lru_cache(maxsize=None)
def _tc_d1(din, dh):
    return pl.pallas_call(
        _tc_d1_body,
        grid=(_NB,),
        in_specs=[
            pl.BlockSpec((_BN, din), lambda i: (i, 0)),
            pl.BlockSpec((din, dh), lambda i: (0, 0)),
            pl.BlockSpec((din, dh), lambda i: (0, 0)),
            pl.BlockSpec((_NW, _BN), lambda i: (0, i)),
        ],
        out_specs=[
            pl.BlockSpec((_BN, dh), lambda i: (i, 0)),
            pl.BlockSpec((_BN, 1), lambda i: (i, 0)),
            pl.BlockSpec((_BN, dh), lambda i: (i, 0)),
        ],
        out_shape=[
            jax.ShapeDtypeStruct((_N, dh), jnp.float32),
            jax.ShapeDtypeStruct((_NP, 1), jnp.float32),
            jax.ShapeDtypeStruct((_N, dh), jnp.float32),
        ],
    )


def _tc_d2_body(z0_ref, t1_ref, dis_ref, b1_ref, w20_ref, w21_ref,
                z20_ref, zs2_ref):
    dh = z0_ref.shape[1]
    t = t1_ref[0, :, :dh] + t1_ref[1, :, :dh]
    dis = dis_ref[...]
    h = jnp.maximum(z0_ref[...] - dis * t + b1_ref[...], 0.0)
    z20_ref[...] = jnp.dot(h, w20_ref[...], preferred_element_type=jnp.float32)
    zs2_ref[...] = (
        jnp.dot(h, w21_ref[...], preferred_element_type=jnp.float32) * dis
    )


@functools.lru_cache(maxsize=None)
def _tc_d2(dh, do):
    return pl.pallas_call(
        _tc_d2_body,
        grid=(_NB,),
        in_specs=[
            pl.BlockSpec((_BN, dh), lambda i: (i, 0)),
            pl.BlockSpec((2, _BN, 128), lambda i: (0, i, 0)),
            pl.BlockSpec((_BN, 1), lambda i: (i, 0)),
            pl.BlockSpec((1, dh), lambda i: (0, 0)),
            pl.BlockSpec((dh, do), lambda i: (0, 0)),
            pl.BlockSpec((dh, do), lambda i: (0, 0)),
        ],
        out_specs=[
            pl.BlockSpec((_BN, do), lambda i: (i, 0)),
            pl.BlockSpec((_BN, do), lambda i: (i, 0)),
        ],
        out_shape=[
            jax.ShapeDtypeStruct((_N, do), jnp.float32),
            jax.ShapeDtypeStruct((_N, do), jnp.float32),
        ],
    )


_BNF = 1024
_NBF = 10


def _tc_fin_body(z20_ref, t2_ref, dis_ref, b2_ref, out_ref):
    do = z20_ref.shape[1]
    t = t2_ref[0, :, :do] + t2_ref[1, :, :do]
    o = z20_ref[...] - dis_ref[...] * t + b2_ref[...]
    m = jnp.max(o, axis=1, keepdims=True)
    lse = jnp.log(jnp.sum(jnp.exp(o - m), axis=1, keepdims=True))
    out_ref[...] = o - m - lse


@functools.lru_cache(maxsize=None)
def _tc_fin(do):
    return pl.pallas_call(
        _tc_fin_body,
        grid=(_NBF,),
        in_specs=[
            pl.BlockSpec((_BNF, do), lambda i: (i, 0)),
            pl.BlockSpec((2, _BNF, 128), lambda i: (0, i, 0)),
            pl.BlockSpec((_BNF, 1), lambda i: (i, 0)),
            pl.BlockSpec((1, do), lambda i: (0, 0)),
        ],
        out_specs=pl.BlockSpec((_BNF, do), lambda i: (i, 0)),
        out_shape=jax.ShapeDtypeStruct((_N, do), jnp.float32),
    )


def kernel(x, edge_index, W10, W11, b1, W20, W21, b2):
    din = x.shape[1]
    dh = W10.shape[1]
    do = W20.shape[1]

    rowpad, colpad, hist = _sc_pre()(edge_index.reshape(2 * _E))
    z0, dis, zs1 = _tc_d1(din, dh)(x, W10, W11, hist)

    row2d = rowpad.reshape(_NW * _NCH, _CH)
    col2d = colpad.reshape(_NW * _NCH, _CH)
    t1 = _sc_edge(dh)(zs1, row2d, col2d)
    z20, zs2 = _tc_d2(dh, do)(z0, t1, dis, b1.reshape(1, dh), W20, W21)
    t2 = _sc_edge(do)(zs2, row2d, col2d)
    return _tc_fin(do)(z20, t2, dis, b2.reshape(1, do))


# speedup vs baseline: 23.8270x; 1.0201x over previous
import functools

import jax
import jax.numpy as jnp
from jax import lax
from jax.experimental import pallas as pl
from jax.experimental.pallas import tpu as pltpu
from jax.experimental.pallas import tpu_sc as plsc

_N = 10000
_E = 160000
_NW = 32
_EP = _E // _NW
_EPP = 5120
_CH = 512
_NCH = _EPP // _CH
_NP = 10240
_RPT = _NP // 16
_DUMMY = _N
_HISTW = 16
_HISTR = 640


def _mesh():
    return plsc.VectorSubcoreMesh(core_axis_name="c", subcore_axis_name="s")


@functools.lru_cache(maxsize=None)
def _sc_pre():
    def body(ei, rowpad, colpad, hist, row_v, col_v, hist_v):
        c = lax.axis_index("c")
        s = lax.axis_index("s")
        wid = c * 16 + s
        base = wid * _EP
        iota16 = lax.iota(jnp.int32, 16)
        zi = iota16 * 0
        zf = zi.astype(jnp.float32)
        ones = zf + 1.0

        def zh(i, carry):
            hist_v[pl.ds(i * 16, 16)] = zf
            return carry
        lax.fori_loop(0, _HISTR, zh, 0)

        pltpu.sync_copy(ei.at[pl.ds(base, _EP)], row_v.at[pl.ds(0, _EP)])
        pltpu.sync_copy(ei.at[pl.ds(_E + base, _EP)], col_v.at[pl.ds(0, _EP)])
        for k in range(7):
            row_v[pl.ds(_EP + 16 * k, 16)] = zi
            col_v[pl.ds(_EP + 16 * k, 16)] = zi
        row_v[pl.ds(_EPP - 16, 16)] = zi
        col_v[pl.ds(_EPP - 16, 16)] = zi

        def eb(j, carry):
            off = j * 16
            r = row_v[pl.ds(off, 16)]
            cc = col_v[pl.ds(off, 16)]
            self_ = r == cc
            col_v[pl.ds(off, 16)] = jnp.where(self_, _DUMMY, cc)
            di = jnp.where(self_, _HISTR * _HISTW - 1, r)
            plsc.addupdate_scatter(hist_v, [di], ones)
            return carry
        lax.fori_loop(0, _EPP // 16, eb, 0)

        pltpu.sync_copy(row_v, rowpad.at[pl.ds(wid * _EPP, _EPP)])
        pltpu.sync_copy(col_v, colpad.at[pl.ds(wid * _EPP, _EPP)])
        pltpu.sync_copy(hist_v, hist.at[wid])

    return pl.kernel(
        body,
        out_type=(
            jax.ShapeDtypeStruct((_NW * _EPP,), jnp.int32),
            jax.ShapeDtypeStruct((_NW * _EPP,), jnp.int32),
            jax.ShapeDtypeStruct((_NW, _HISTR * _HISTW), jnp.float32),
        ),
        mesh=_mesh(),
        scratch_types=[
            pltpu.VMEM((_EPP,), jnp.int32),
            pltpu.VMEM((_EPP,), jnp.int32),
            pltpu.VMEM((_HISTR * _HISTW,), jnp.float32),
        ],
        compiler_params=pltpu.CompilerParams(needs_layout_passes=False),
    )


@functools.lru_cache(maxsize=None)
def _sc_edge(D):
    def body(z, rowi, coli, t_out, ri_v, ci_v, g0, g1, g2, zb_v, t_sh,
             sg0, sg1, sg2, ss0, ss1, ss2):
        c = lax.axis_index("c")
        s = lax.axis_index("s")
        wid = c * 16 + s
        zf = lax.iota(jnp.int32, 16).astype(jnp.float32) * 0.0

        def zb(i, carry):
            for k in range(D // 16):
                zb_v[i, pl.ds(16 * k, 16)] = zf
            return carry
        lax.fori_loop(0, _RPT, zb, 0)
        pltpu.sync_copy(zb_v, t_sh.at[pl.ds(s * _RPT, _RPT)])

        pltpu.sync_copy(rowi.at[pl.ds(wid * _NCH, _NCH)], ri_v)
        pltpu.sync_copy(coli.at[pl.ds(wid * _NCH, _NCH)], ci_v)
        plsc.subcore_barrier()

        bufs = [(g0, sg0, ss0), (g1, sg1, ss1), (g2, sg2, ss2)]

        def start_g(j, b):
            pltpu.async_copy(z.at[ri_v.at[j]], bufs[b][0], bufs[b][1])

        def wait_g(b):
            pltpu.make_async_copy(z.at[ri_v.at[0]], bufs[b][0], bufs[b][1]).wait()

        def start_s(j, b):
            pltpu.async_copy(bufs[b][0], t_sh.at[ci_v.at[j]], bufs[b][2],
                             add=True)

        def wait_s(b):
            pltpu.make_async_copy(
                bufs[b][0], t_sh.at[ci_v.at[0]], bufs[b][2]).wait()

        for j in range(min(3, _NCH)):
            start_g(j, j)
        for j in range(_NCH):
            b = j % 3
            wait_g(b)
            start_s(j, b)
            if j + 3 < _NCH:
                wait_s(b)
                start_g(j + 3, b)
        for j in range(max(0, _NCH - 3), _NCH):
            wait_s(j % 3)

        plsc.subcore_barrier()
        pltpu.sync_copy(t_sh.at[pl.ds(s * _RPT, _RPT)], zb_v)
        pltpu.sync_copy(zb_v, t_out.at[c, pl.ds(s * _RPT, _RPT), pl.ds(0, D)])

    return pl.kernel(
        body,
        out_type=jax.ShapeDtypeStruct((2, _NP, 128), jnp.float32),
        mesh=_mesh(),
        scratch_types=[
            pltpu.VMEM((_NCH, _CH), jnp.int32),
            pltpu.VMEM((_NCH, _CH), jnp.int32),
            pltpu.VMEM((_CH, D), jnp.float32),
            pltpu.VMEM((_CH, D), jnp.float32),
            pltpu.VMEM((_CH, D), jnp.float32),
            pltpu.VMEM((_RPT, D), jnp.float32),
            pltpu.VMEM_SHARED((_NP, D), jnp.float32),
            pltpu.SemaphoreType.DMA,
            pltpu.SemaphoreType.DMA,
            pltpu.SemaphoreType.DMA,
            pltpu.SemaphoreType.DMA,
            pltpu.SemaphoreType.DMA,
            pltpu.SemaphoreType.DMA,
        ],
        compiler_params=pltpu.CompilerParams(
            needs_layout_passes=False, use_tc_tiling_on_sc=False
        ),
    )


_BN = 1024
_NB = 10


def _tc_d1_body(x_ref, w10_ref, w11_ref, hist_ref, z0_ref, dis_ref, zs1_ref):
    deg = jnp.sum(hist_ref[...], axis=0, keepdims=True)
    safe = jnp.where(deg > 0, deg, 1.0)
    dis = jnp.where(deg > 0, lax.rsqrt(safe), 0.0)
    dis_col = dis.reshape(dis.shape[1], 1)
    dis_ref[...] = dis_col
    xb = x_ref[...]
    z0_ref[...] = jnp.dot(xb, w10_ref[...], preferred_element_type=jnp.float32)
    zs1_ref[...] = (
        jnp.dot(xb, w11_ref[...], preferred_element_type=jnp.float32) * dis_col
    )


@functools.lru_cache(maxsize=None)
def _tc_d1(din, dh):
    return pl.pallas_call(
        _tc_d1_body,
        grid=(_NB,),
        in_specs=[
            pl.BlockSpec((_BN, din), lambda i: (i, 0)),
            pl.BlockSpec((din, dh), lambda i: (0, 0)),
            pl.BlockSpec((din, dh), lambda i: (0, 0)),
            pl.BlockSpec((_NW, _BN), lambda i: (0, i)),
        ],
        out_specs=[
            pl.BlockSpec((_BN, dh), lambda i: (i, 0)),
            pl.BlockSpec((_BN, 1), lambda i: (i, 0)),
            pl.BlockSpec((_BN, dh), lambda i: (i, 0)),
        ],
        out_shape=[
            jax.ShapeDtypeStruct((_N, dh), jnp.float32),
            jax.ShapeDtypeStruct((_NP, 1), jnp.float32),
            jax.ShapeDtypeStruct((_N, dh), jnp.float32),
        ],
    )


def _tc_d2_body(z0_ref, t1_ref, dis_ref, b1_ref, w20_ref, w21_ref,
                z20_ref, zs2_ref):
    dh = z0_ref.shape[1]
    t = t1_ref[0, :, :dh] + t1_ref[1, :, :dh]
    dis = dis_ref[...]
    h = jnp.maximum(z0_ref[...] - dis * t + b1_ref[...], 0.0)
    z20_ref[...] = jnp.dot(h, w20_ref[...], preferred_element_type=jnp.float32)
    zs2_ref[...] = (
        jnp.dot(h, w21_ref[...], preferred_element_type=jnp.float32) * dis
    )


@functools.lru_cache(maxsize=None)
def _tc_d2(dh, do):
    return pl.pallas_call(
        _tc_d2_body,
        grid=(_NB,),
        in_specs=[
            pl.BlockSpec((_BN, dh), lambda i: (i, 0)),
            pl.BlockSpec((2, _BN, 128), lambda i: (0, i, 0)),
            pl.BlockSpec((_BN, 1), lambda i: (i, 0)),
            pl.BlockSpec((1, dh), lambda i: (0, 0)),
            pl.BlockSpec((dh, do), lambda i: (0, 0)),
            pl.BlockSpec((dh, do), lambda i: (0, 0)),
        ],
        out_specs=[
            pl.BlockSpec((_BN, do), lambda i: (i, 0)),
            pl.BlockSpec((_BN, do), lambda i: (i, 0)),
        ],
        out_shape=[
            jax.ShapeDtypeStruct((_N, do), jnp.float32),
            jax.ShapeDtypeStruct((_N, do), jnp.float32),
        ],
    )


_BNF = 1024
_NBF = 10


def _tc_fin_body(z20_ref, t2_ref, dis_ref, b2_ref, out_ref):
    do = z20_ref.shape[1]
    t = t2_ref[0, :, :do] + t2_ref[1, :, :do]
    o = z20_ref[...] - dis_ref[...] * t + b2_ref[...]
    m = jnp.max(o, axis=1, keepdims=True)
    lse = jnp.log(jnp.sum(jnp.exp(o - m), axis=1, keepdims=True))
    out_ref[...] = o - m - lse


@functools.lru_cache(maxsize=None)
def _tc_fin(do):
    return pl.pallas_call(
        _tc_fin_body,
        grid=(_NBF,),
        in_specs=[
            pl.BlockSpec((_BNF, do), lambda i: (i, 0)),
            pl.BlockSpec((2, _BNF, 128), lambda i: (0, i, 0)),
            pl.BlockSpec((_BNF, 1), lambda i: (i, 0)),
            pl.BlockSpec((1, do), lambda i: (0, 0)),
        ],
        out_specs=pl.BlockSpec((_BNF, do), lambda i: (i, 0)),
        out_shape=jax.ShapeDtypeStruct((_N, do), jnp.float32),
    )


def kernel(x, edge_index, W10, W11, b1, W20, W21, b2):
    din = x.shape[1]
    dh = W10.shape[1]
    do = W20.shape[1]

    rowpad, colpad, hist = _sc_pre()(edge_index.reshape(2 * _E))
    z0, dis, zs1 = _tc_d1(din, dh)(x, W10, W11, hist)

    row2d = rowpad.reshape(_NW * _NCH, _CH)
    col2d = colpad.reshape(_NW * _NCH, _CH)
    t1 = _sc_edge(dh)(zs1, row2d, col2d)
    z20, zs2 = _tc_d2(dh, do)(z0, t1, dis, b1.reshape(1, dh), W20, W21)
    t2 = _sc_edge(do)(zs2, row2d, col2d)
    return _tc_fin(do)(z20, t2, dis, b2.reshape(1, do))


# speedup vs baseline: 23.9615x; 1.0056x over previous
import functools

import jax
import jax.numpy as jnp
from jax import lax
from jax.experimental import pallas as pl
from jax.experimental.pallas import tpu as pltpu
from jax.experimental.pallas import tpu_sc as plsc

_N = 10000
_E = 160000
_NW = 32
_EP = _E // _NW
_EPP = 5120
_CH = 512
_NCH = _EPP // _CH
_NP = 10240
_RPT = _NP // 16
_DUMMY = _N
_HISTW = 16
_HISTR = 640


def _mesh():
    return plsc.VectorSubcoreMesh(core_axis_name="c", subcore_axis_name="s")


@functools.lru_cache(maxsize=None)
def _sc_pre():
    def body(ei, rowpad, colpad, hist, row_v, col_v, hist_v):
        c = lax.axis_index("c")
        s = lax.axis_index("s")
        wid = c * 16 + s
        base = wid * _EP
        iota16 = lax.iota(jnp.int32, 16)
        zi = iota16 * 0
        zf = zi.astype(jnp.float32)
        ones = zf + 1.0

        def zh(i, carry):
            hist_v[pl.ds(i * 16, 16)] = zf
            return carry
        lax.fori_loop(0, _HISTR, zh, 0)

        pltpu.sync_copy(ei.at[pl.ds(base, _EP)], row_v.at[pl.ds(0, _EP)])
        pltpu.sync_copy(ei.at[pl.ds(_E + base, _EP)], col_v.at[pl.ds(0, _EP)])
        for k in range(7):
            row_v[pl.ds(_EP + 16 * k, 16)] = zi
            col_v[pl.ds(_EP + 16 * k, 16)] = zi
        row_v[pl.ds(_EPP - 16, 16)] = zi
        col_v[pl.ds(_EPP - 16, 16)] = zi

        def eb(j, carry):
            off = j * 16
            r = row_v[pl.ds(off, 16)]
            cc = col_v[pl.ds(off, 16)]
            self_ = r == cc
            col_v[pl.ds(off, 16)] = jnp.where(self_, _DUMMY, cc)
            di = jnp.where(self_, _HISTR * _HISTW - 1, r)
            plsc.addupdate_scatter(hist_v, [di], ones)
            return carry
        lax.fori_loop(0, _EPP // 16, eb, 0)

        pltpu.sync_copy(row_v, rowpad.at[pl.ds(wid * _EPP, _EPP)])
        pltpu.sync_copy(col_v, colpad.at[pl.ds(wid * _EPP, _EPP)])
        pltpu.sync_copy(hist_v, hist.at[wid])

    return pl.kernel(
        body,
        out_type=(
            jax.ShapeDtypeStruct((_NW * _EPP,), jnp.int32),
            jax.ShapeDtypeStruct((_NW * _EPP,), jnp.int32),
            jax.ShapeDtypeStruct((_NW, _HISTR * _HISTW), jnp.float32),
        ),
        mesh=_mesh(),
        scratch_types=[
            pltpu.VMEM((_EPP,), jnp.int32),
            pltpu.VMEM((_EPP,), jnp.int32),
            pltpu.VMEM((_HISTR * _HISTW,), jnp.float32),
        ],
        compiler_params=pltpu.CompilerParams(needs_layout_passes=False),
    )


@functools.lru_cache(maxsize=None)
def _sc_edge(D):
    def body(z, rowi, coli, t_out, ri_v, ci_v, g0, g1, g2, g3, zb_v, t_sh,
             sg0, sg1, sg2, sg3, ss0, ss1, ss2, ss3):
        c = lax.axis_index("c")
        s = lax.axis_index("s")
        wid = c * 16 + s
        zf = lax.iota(jnp.int32, 16).astype(jnp.float32) * 0.0

        def zb(i, carry):
            for k in range(D // 16):
                zb_v[i, pl.ds(16 * k, 16)] = zf
            return carry
        lax.fori_loop(0, _RPT, zb, 0)
        pltpu.sync_copy(zb_v, t_sh.at[pl.ds(s * _RPT, _RPT)])

        pltpu.sync_copy(rowi.at[pl.ds(wid * _NCH, _NCH)], ri_v)
        pltpu.sync_copy(coli.at[pl.ds(wid * _NCH, _NCH)], ci_v)
        plsc.subcore_barrier()

        bufs = [(g0, sg0, ss0), (g1, sg1, ss1), (g2, sg2, ss2), (g3, sg3, ss3)]
        nb = len(bufs)

        def start_g(j, b):
            pltpu.async_copy(z.at[ri_v.at[j]], bufs[b][0], bufs[b][1])

        def wait_g(b):
            pltpu.make_async_copy(z.at[ri_v.at[0]], bufs[b][0], bufs[b][1]).wait()

        def start_s(j, b):
            pltpu.async_copy(bufs[b][0], t_sh.at[ci_v.at[j]], bufs[b][2],
                             add=True)

        def wait_s(b):
            pltpu.make_async_copy(
                bufs[b][0], t_sh.at[ci_v.at[0]], bufs[b][2]).wait()

        for j in range(min(nb, _NCH)):
            start_g(j, j)
        for j in range(_NCH):
            b = j % nb
            wait_g(b)
            start_s(j, b)
            if j + nb < _NCH:
                wait_s(b)
                start_g(j + nb, b)
        for j in range(max(0, _NCH - nb), _NCH):
            wait_s(j % nb)

        plsc.subcore_barrier()
        pltpu.sync_copy(t_sh.at[pl.ds(s * _RPT, _RPT)], zb_v)
        pltpu.sync_copy(zb_v, t_out.at[c, pl.ds(s * _RPT, _RPT), pl.ds(0, D)])

    return pl.kernel(
        body,
        out_type=jax.ShapeDtypeStruct((2, _NP, 128), jnp.float32),
        mesh=_mesh(),
        scratch_types=[
            pltpu.VMEM((_NCH, _CH), jnp.int32),
            pltpu.VMEM((_NCH, _CH), jnp.int32),
            pltpu.VMEM((_CH, D), jnp.float32),
            pltpu.VMEM((_CH, D), jnp.float32),
            pltpu.VMEM((_CH, D), jnp.float32),
            pltpu.VMEM((_CH, D), jnp.float32),
            pltpu.VMEM((_RPT, D), jnp.float32),
            pltpu.VMEM_SHARED((_NP, D), jnp.float32),
            pltpu.SemaphoreType.DMA,
            pltpu.SemaphoreType.DMA,
            pltpu.SemaphoreType.DMA,
            pltpu.SemaphoreType.DMA,
            pltpu.SemaphoreType.DMA,
            pltpu.SemaphoreType.DMA,
            pltpu.SemaphoreType.DMA,
            pltpu.SemaphoreType.DMA,
        ],
        compiler_params=pltpu.CompilerParams(
            needs_layout_passes=False, use_tc_tiling_on_sc=False
        ),
    )


_BN = 1024
_NB = 10


def _tc_d1_body(x_ref, w10_ref, w11_ref, hist_ref, z0_ref, dis_ref, zs1_ref):
    deg = jnp.sum(hist_ref[...], axis=0, keepdims=True)
    safe = jnp.where(deg > 0, deg, 1.0)
    dis = jnp.where(deg > 0, lax.rsqrt(safe), 0.0)
    dis_col = dis.reshape(dis.shape[1], 1)
    dis_ref[...] = dis_col
    xb = x_ref[...]
    z0_ref[...] = jnp.dot(xb, w10_ref[...], preferred_element_type=jnp.float32)
    zs1_ref[...] = (
        jnp.dot(xb, w11_ref[...], preferred_element_type=jnp.float32) * dis_col
    )


@functools.lru_cache(maxsize=None)
def _tc_d1(din, dh):
    return pl.pallas_call(
        _tc_d1_body,
        grid=(_NB,),
        in_specs=[
            pl.BlockSpec((_BN, din), lambda i: (i, 0)),
            pl.BlockSpec((din, dh), lambda i: (0, 0)),
            pl.BlockSpec((din, dh), lambda i: (0, 0)),
            pl.BlockSpec((_NW, _BN), lambda i: (0, i)),
        ],
        out_specs=[
            pl.BlockSpec((_BN, dh), lambda i: (i, 0)),
            pl.BlockSpec((_BN, 1), lambda i: (i, 0)),
            pl.BlockSpec((_BN, dh), lambda i: (i, 0)),
        ],
        out_shape=[
            jax.ShapeDtypeStruct((_N, dh), jnp.float32),
            jax.ShapeDtypeStruct((_NP, 1), jnp.float32),
            jax.ShapeDtypeStruct((_N, dh), jnp.float32),
        ],
    )


def _tc_d2_body(z0_ref, t1_ref, dis_ref, b1_ref, w20_ref, w21_ref,
                z20_ref, zs2_ref):
    dh = z0_ref.shape[1]
    t = t1_ref[0, :, :dh] + t1_ref[1, :, :dh]
    dis = dis_ref[...]
    h = jnp.maximum(z0_ref[...] - dis * t + b1_ref[...], 0.0)
    z20_ref[...] = jnp.dot(h, w20_ref[...], preferred_element_type=jnp.float32)
    zs2_ref[...] = (
        jnp.dot(h, w21_ref[...], preferred_element_type=jnp.float32) * dis
    )


@functools.lru_cache(maxsize=None)
def _tc_d2(dh, do):
    return pl.pallas_call(
        _tc_d2_body,
        grid=(_NB,),
        in_specs=[
            pl.BlockSpec((_BN, dh), lambda i: (i, 0)),
            pl.BlockSpec((2, _BN, 128), lambda i: (0, i, 0)),
            pl.BlockSpec((_BN, 1), lambda i: (i, 0)),
            pl.BlockSpec((1, dh), lambda i: (0, 0)),
            pl.BlockSpec((dh, do), lambda i: (0, 0)),
            pl.BlockSpec((dh, do), lambda i: (0, 0)),
        ],
        out_specs=[
            pl.BlockSpec((_BN, do), lambda i: (i, 0)),
            pl.BlockSpec((_BN, do), lambda i: (i, 0)),
        ],
        out_shape=[
            jax.ShapeDtypeStruct((_N, do), jnp.float32),
            jax.ShapeDtypeStruct((_N, do), jnp.float32),
        ],
    )


_BNF = 1024
_NBF = 10


def _tc_fin_body(z20_ref, t2_ref, dis_ref, b2_ref, out_ref):
    do = z20_ref.shape[1]
    t = t2_ref[0, :, :do] + t2_ref[1, :, :do]
    o = z20_ref[...] - dis_ref[...] * t + b2_ref[...]
    m = jnp.max(o, axis=1, keepdims=True)
    lse = jnp.log(jnp.sum(jnp.exp(o - m), axis=1, keepdims=True))
    out_ref[...] = o - m - lse


@functools.lru_cache(maxsize=None)
def _tc_fin(do):
    return pl.pallas_call(
        _tc_fin_body,
        grid=(_NBF,),
        in_specs=[
            pl.BlockSpec((_BNF, do), lambda i: (i, 0)),
            pl.BlockSpec((2, _BNF, 128), lambda i: (0, i, 0)),
            pl.BlockSpec((_BNF, 1), lambda i: (i, 0)),
            pl.BlockSpec((1, do), lambda i: (0, 0)),
        ],
        out_specs=pl.BlockSpec((_BNF, do), lambda i: (i, 0)),
        out_shape=jax.ShapeDtypeStruct((_N, do), jnp.float32),
    )


def kernel(x, edge_index, W10, W11, b1, W20, W21, b2):
    din = x.shape[1]
    dh = W10.shape[1]
    do = W20.shape[1]

    rowpad, colpad, hist = _sc_pre()(edge_index.reshape(2 * _E))
    z0, dis, zs1 = _tc_d1(din, dh)(x, W10, W11, hist)

    row2d = rowpad.reshape(_NW * _NCH, _CH)
    col2d = colpad.reshape(_NW * _NCH, _CH)
    t1 = _sc_edge(dh)(zs1, row2d, col2d)
    z20, zs2 = _tc_d2(dh, do)(z0, t1, dis, b1.reshape(1, dh), W20, W21)
    t2 = _sc_edge(do)(zs2, row2d, col2d)
    return _tc_fin(do)(z20, t2, dis, b2.reshape(1, do))


# speedup vs baseline: 24.0413x; 1.0033x over previous
import functools

import jax
import jax.numpy as jnp
from jax import lax
from jax.experimental import pallas as pl
from jax.experimental.pallas import tpu as pltpu
from jax.experimental.pallas import tpu_sc as plsc

_N = 10000
_E = 160000
_NW = 32
_EP = _E // _NW
_EPP = 5120
_CH = 512
_NCH = _EPP // _CH
_NP = 10240
_RPT = _NP // 16
_DUMMY = _N
_HISTW = 16
_HISTR = 640


def _mesh():
    return plsc.VectorSubcoreMesh(core_axis_name="c", subcore_axis_name="s")


@functools.lru_cache(maxsize=None)
def _sc_pre():
    def body(ei, rowpad, colpad, hist, row_v, col_v, hist_v):
        c = lax.axis_index("c")
        s = lax.axis_index("s")
        wid = c * 16 + s
        base = wid * _EP
        iota16 = lax.iota(jnp.int32, 16)
        zi = iota16 * 0
        zf = zi.astype(jnp.float32)
        ones = zf + 1.0

        def zh(i, carry):
            hist_v[pl.ds(i * 16, 16)] = zf
            return carry
        lax.fori_loop(0, _HISTR, zh, 0)

        pltpu.sync_copy(ei.at[pl.ds(base, _EP)], row_v.at[pl.ds(0, _EP)])
        pltpu.sync_copy(ei.at[pl.ds(_E + base, _EP)], col_v.at[pl.ds(0, _EP)])
        for k in range(7):
            row_v[pl.ds(_EP + 16 * k, 16)] = zi
            col_v[pl.ds(_EP + 16 * k, 16)] = zi
        row_v[pl.ds(_EPP - 16, 16)] = zi
        col_v[pl.ds(_EPP - 16, 16)] = zi

        def eb(j, carry):
            off = j * 16
            r = row_v[pl.ds(off, 16)]
            cc = col_v[pl.ds(off, 16)]
            self_ = r == cc
            col_v[pl.ds(off, 16)] = jnp.where(self_, _DUMMY, cc)
            di = jnp.where(self_, _HISTR * _HISTW - 1, r)
            plsc.addupdate_scatter(hist_v, [di], ones)
            return carry
        lax.fori_loop(0, _EPP // 16, eb, 0)

        pltpu.sync_copy(row_v, rowpad.at[pl.ds(wid * _EPP, _EPP)])
        pltpu.sync_copy(col_v, colpad.at[pl.ds(wid * _EPP, _EPP)])
        pltpu.sync_copy(hist_v, hist.at[wid])

    return pl.kernel(
        body,
        out_type=(
            jax.ShapeDtypeStruct((_NW * _EPP,), jnp.int32),
            jax.ShapeDtypeStruct((_NW * _EPP,), jnp.int32),
            jax.ShapeDtypeStruct((_NW, _HISTR * _HISTW), jnp.float32),
        ),
        mesh=_mesh(),
        scratch_types=[
            pltpu.VMEM((_EPP,), jnp.int32),
            pltpu.VMEM((_EPP,), jnp.int32),
            pltpu.VMEM((_HISTR * _HISTW,), jnp.float32),
        ],
        compiler_params=pltpu.CompilerParams(needs_layout_passes=False),
    )


@functools.lru_cache(maxsize=None)
def _sc_edge(D):
    def body(z, rowi, coli, t_out, ri_v, ci_v, g0, g1, g2, g3, zb_v, t_sh,
             sg0, sg1, sg2, sg3, ss0, ss1, ss2, ss3):
        c = lax.axis_index("c")
        s = lax.axis_index("s")
        wid = c * 16 + s
        zf = lax.iota(jnp.int32, 16).astype(jnp.float32) * 0.0

        def zb(i, carry):
            for k in range(D // 16):
                zb_v[i, pl.ds(16 * k, 16)] = zf
            return carry
        lax.fori_loop(0, _RPT, zb, 0)
        pltpu.sync_copy(zb_v, t_sh.at[pl.ds(s * _RPT, _RPT)])

        pltpu.sync_copy(rowi.at[pl.ds(wid * _NCH, _NCH)], ri_v)
        pltpu.sync_copy(coli.at[pl.ds(wid * _NCH, _NCH)], ci_v)
        plsc.subcore_barrier()

        bufs = [(g0, sg0, ss0), (g1, sg1, ss1), (g2, sg2, ss2), (g3, sg3, ss3)]
        nb = len(bufs)

        def start_g(j, b):
            pltpu.async_copy(z.at[ri_v.at[j]], bufs[b][0], bufs[b][1])

        def wait_g(b):
            pltpu.make_async_copy(z.at[ri_v.at[0]], bufs[b][0], bufs[b][1]).wait()

        def start_s(j, b):
            pltpu.async_copy(bufs[b][0], t_sh.at[ci_v.at[j]], bufs[b][2],
                             add=True)

        def wait_s(b):
            pltpu.make_async_copy(
                bufs[b][0], t_sh.at[ci_v.at[0]], bufs[b][2]).wait()

        for j in range(min(nb, _NCH)):
            start_g(j, j)
        for j in range(_NCH):
            b = j % nb
            wait_g(b)
            start_s(j, b)
            if j + nb < _NCH:
                wait_s(b)
                start_g(j + nb, b)
        for j in range(max(0, _NCH - nb), _NCH):
            wait_s(j % nb)

        plsc.subcore_barrier()
        pltpu.sync_copy(t_sh.at[pl.ds(s * _RPT, _RPT)], zb_v)
        pltpu.sync_copy(zb_v,
                        t_out.at[pl.ds(s * _RPT, _RPT), pl.ds(c * D, D)])

    return pl.kernel(
        body,
        out_type=jax.ShapeDtypeStruct((_NP, 128), jnp.float32),
        mesh=_mesh(),
        scratch_types=[
            pltpu.VMEM((_NCH, _CH), jnp.int32),
            pltpu.VMEM((_NCH, _CH), jnp.int32),
            pltpu.VMEM((_CH, D), jnp.float32),
            pltpu.VMEM((_CH, D), jnp.float32),
            pltpu.VMEM((_CH, D), jnp.float32),
            pltpu.VMEM((_CH, D), jnp.float32),
            pltpu.VMEM((_RPT, D), jnp.float32),
            pltpu.VMEM_SHARED((_NP, D), jnp.float32),
            pltpu.SemaphoreType.DMA,
            pltpu.SemaphoreType.DMA,
            pltpu.SemaphoreType.DMA,
            pltpu.SemaphoreType.DMA,
            pltpu.SemaphoreType.DMA,
            pltpu.SemaphoreType.DMA,
            pltpu.SemaphoreType.DMA,
            pltpu.SemaphoreType.DMA,
        ],
        compiler_params=pltpu.CompilerParams(
            needs_layout_passes=False, use_tc_tiling_on_sc=False
        ),
    )


_BN = 1024
_NB = 10


def _tc_d1_body(x_ref, w10_ref, w11_ref, hist_ref, z0_ref, dis_ref, zs1_ref):
    deg = jnp.sum(hist_ref[...], axis=0, keepdims=True)
    safe = jnp.where(deg > 0, deg, 1.0)
    dis = jnp.where(deg > 0, lax.rsqrt(safe), 0.0)
    dis_col = dis.reshape(dis.shape[1], 1)
    dis_ref[...] = dis_col
    xb = x_ref[...]
    z0_ref[...] = jnp.dot(xb, w10_ref[...], preferred_element_type=jnp.float32)
    zs1_ref[...] = (
        jnp.dot(xb, w11_ref[...], preferred_element_type=jnp.float32) * dis_col
    )


@functools.lru_cache(maxsize=None)
def _tc_d1(din, dh):
    return pl.pallas_call(
        _tc_d1_body,
        grid=(_NB,),
        in_specs=[
            pl.BlockSpec((_BN, din), lambda i: (i, 0)),
            pl.BlockSpec((din, dh), lambda i: (0, 0)),
            pl.BlockSpec((din, dh), lambda i: (0, 0)),
            pl.BlockSpec((_NW, _BN), lambda i: (0, i)),
        ],
        out_specs=[
            pl.BlockSpec((_BN, dh), lambda i: (i, 0)),
            pl.BlockSpec((_BN, 1), lambda i: (i, 0)),
            pl.BlockSpec((_BN, dh), lambda i: (i, 0)),
        ],
        out_shape=[
            jax.ShapeDtypeStruct((_N, dh), jnp.float32),
            jax.ShapeDtypeStruct((_NP, 1), jnp.float32),
            jax.ShapeDtypeStruct((_N, dh), jnp.float32),
        ],
    )


def _tc_d2_body(z0_ref, t1_ref, dis_ref, b1_ref, w20_ref, w21_ref,
                z20_ref, zs2_ref):
    dh = z0_ref.shape[1]
    t = t1_ref[:, :dh] + t1_ref[:, dh:2 * dh]
    dis = dis_ref[...]
    h = jnp.maximum(z0_ref[...] - dis * t + b1_ref[...], 0.0)
    z20_ref[...] = jnp.dot(h, w20_ref[...], preferred_element_type=jnp.float32)
    zs2_ref[...] = (
        jnp.dot(h, w21_ref[...], preferred_element_type=jnp.float32) * dis
    )


@functools.lru_cache(maxsize=None)
def _tc_d2(dh, do):
    return pl.pallas_call(
        _tc_d2_body,
        grid=(_NB,),
        in_specs=[
            pl.BlockSpec((_BN, dh), lambda i: (i, 0)),
            pl.BlockSpec((_BN, 128), lambda i: (i, 0)),
            pl.BlockSpec((_BN, 1), lambda i: (i, 0)),
            pl.BlockSpec((1, dh), lambda i: (0, 0)),
            pl.BlockSpec((dh, do), lambda i: (0, 0)),
            pl.BlockSpec((dh, do), lambda i: (0, 0)),
        ],
        out_specs=[
            pl.BlockSpec((_BN, do), lambda i: (i, 0)),
            pl.BlockSpec((_BN, do), lambda i: (i, 0)),
        ],
        out_shape=[
            jax.ShapeDtypeStruct((_N, do), jnp.float32),
            jax.ShapeDtypeStruct((_N, do), jnp.float32),
        ],
    )


_BNF = 1024
_NBF = 10


def _tc_fin_body(z20_ref, t2_ref, dis_ref, b2_ref, out_ref):
    do = z20_ref.shape[1]
    t = t2_ref[:, :do] + t2_ref[:, do:2 * do]
    o = z20_ref[...] - dis_ref[...] * t + b2_ref[...]
    m = jnp.max(o, axis=1, keepdims=True)
    lse = jnp.log(jnp.sum(jnp.exp(o - m), axis=1, keepdims=True))
    out_ref[...] = o - m - lse


@functools.lru_cache(maxsize=None)
def _tc_fin(do):
    return pl.pallas_call(
        _tc_fin_body,
        grid=(_NBF,),
        in_specs=[
            pl.BlockSpec((_BNF, do), lambda i: (i, 0)),
            pl.BlockSpec((_BNF, 128), lambda i: (i, 0)),
            pl.BlockSpec((_BNF, 1), lambda i: (i, 0)),
            pl.BlockSpec((1, do), lambda i: (0, 0)),
        ],
        out_specs=pl.BlockSpec((_BNF, do), lambda i: (i, 0)),
        out_shape=jax.ShapeDtypeStruct((_N, do), jnp.float32),
    )


def kernel(x, edge_index, W10, W11, b1, W20, W21, b2):
    din = x.shape[1]
    dh = W10.shape[1]
    do = W20.shape[1]

    rowpad, colpad, hist = _sc_pre()(edge_index.reshape(2 * _E))
    z0, dis, zs1 = _tc_d1(din, dh)(x, W10, W11, hist)

    row2d = rowpad.reshape(_NW * _NCH, _CH)
    col2d = colpad.reshape(_NW * _NCH, _CH)
    t1 = _sc_edge(dh)(zs1, row2d, col2d)
    z20, zs2 = _tc_d2(dh, do)(z0, t1, dis, b1.reshape(1, dh), W20, W21)
    t2 = _sc_edge(do)(zs2, row2d, col2d)
    return _tc_fin(do)(z20, t2, dis, b2.reshape(1, do))


# speedup vs baseline: 24.5298x; 1.0203x over previous
import functools

import jax
import jax.numpy as jnp
from jax import lax
from jax.experimental import pallas as pl
from jax.experimental.pallas import tpu as pltpu
from jax.experimental.pallas import tpu_sc as plsc

_N = 10000
_E = 160000
_NW = 32
_EP = _E // _NW
_EPP = 5120
_CH = 512
_NCH = _EPP // _CH
_NP = 10240
_RPT = _NP // 16
_DUMMY = _N
_HISTW = 16
_HISTR = 640


def _mesh():
    return plsc.VectorSubcoreMesh(core_axis_name="c", subcore_axis_name="s")


@functools.lru_cache(maxsize=None)
def _sc_pre():
    def body(ei, rowpad, colpad, hist, row_v, col_v, hist_v):
        c = lax.axis_index("c")
        s = lax.axis_index("s")
        wid = c * 16 + s
        base = wid * _EP
        iota16 = lax.iota(jnp.int32, 16)
        zi = iota16 * 0
        zf = zi.astype(jnp.float32)
        ones = zf + 1.0

        def zh(i, carry):
            hist_v[pl.ds(i * 16, 16)] = zf
            return carry
        lax.fori_loop(0, _HISTR, zh, 0)

        pltpu.sync_copy(ei.at[pl.ds(base, _EP)], row_v.at[pl.ds(0, _EP)])
        pltpu.sync_copy(ei.at[pl.ds(_E + base, _EP)], col_v.at[pl.ds(0, _EP)])
        for k in range(7):
            row_v[pl.ds(_EP + 16 * k, 16)] = zi
            col_v[pl.ds(_EP + 16 * k, 16)] = zi
        row_v[pl.ds(_EPP - 16, 16)] = zi
        col_v[pl.ds(_EPP - 16, 16)] = zi

        def eb(j, carry):
            off = j * 16
            r = row_v[pl.ds(off, 16)]
            cc = col_v[pl.ds(off, 16)]
            self_ = r == cc
            col_v[pl.ds(off, 16)] = jnp.where(self_, _DUMMY, cc)
            di = jnp.where(self_, _HISTR * _HISTW - 1, r)
            plsc.addupdate_scatter(hist_v, [di], ones)
            return carry
        lax.fori_loop(0, _EPP // 16, eb, 0)

        pltpu.sync_copy(row_v, rowpad.at[pl.ds(wid * _EPP, _EPP)])
        pltpu.sync_copy(col_v, colpad.at[pl.ds(wid * _EPP, _EPP)])
        pltpu.sync_copy(hist_v, hist.at[wid])

    return pl.kernel(
        body,
        out_type=(
            jax.ShapeDtypeStruct((_NW * _EPP,), jnp.int32),
            jax.ShapeDtypeStruct((_NW * _EPP,), jnp.int32),
            jax.ShapeDtypeStruct((_NW, _HISTR * _HISTW), jnp.float32),
        ),
        mesh=_mesh(),
        scratch_types=[
            pltpu.VMEM((_EPP,), jnp.int32),
            pltpu.VMEM((_EPP,), jnp.int32),
            pltpu.VMEM((_HISTR * _HISTW,), jnp.float32),
        ],
        compiler_params=pltpu.CompilerParams(needs_layout_passes=False),
    )


@functools.lru_cache(maxsize=None)
def _sc_edge(D):
    def body(z, rowi, coli, t_out, ri_v, ci_v, g0, g1, g2, g3, zb_v, t_sh,
             sg0, sg1, sg2, sg3, ss0, ss1, ss2, ss3):
        c = lax.axis_index("c")
        s = lax.axis_index("s")
        wid = c * 16 + s
        zf = lax.iota(jnp.int32, 16).astype(jnp.float32) * 0.0

        pltpu.async_copy(rowi.at[pl.ds(wid * _NCH, _NCH)], ri_v, sg0)
        pltpu.async_copy(coli.at[pl.ds(wid * _NCH, _NCH)], ci_v, sg1)

        def zb(i, carry):
            for k in range(D // 16):
                zb_v[i, pl.ds(16 * k, 16)] = zf
            return carry
        lax.fori_loop(0, _RPT, zb, 0)
        pltpu.sync_copy(zb_v, t_sh.at[pl.ds(s * _RPT, _RPT)])

        pltpu.make_async_copy(
            rowi.at[pl.ds(wid * _NCH, _NCH)], ri_v, sg0).wait()
        pltpu.make_async_copy(
            coli.at[pl.ds(wid * _NCH, _NCH)], ci_v, sg1).wait()
        plsc.subcore_barrier()

        bufs = [(g0, sg0, ss0), (g1, sg1, ss1), (g2, sg2, ss2), (g3, sg3, ss3)]
        nb = len(bufs)

        def start_g(j, b):
            pltpu.async_copy(z.at[ri_v.at[j]], bufs[b][0], bufs[b][1])

        def wait_g(b):
            pltpu.make_async_copy(z.at[ri_v.at[0]], bufs[b][0], bufs[b][1]).wait()

        def start_s(j, b):
            pltpu.async_copy(bufs[b][0], t_sh.at[ci_v.at[j]], bufs[b][2],
                             add=True)

        def wait_s(b):
            pltpu.make_async_copy(
                bufs[b][0], t_sh.at[ci_v.at[0]], bufs[b][2]).wait()

        for j in range(min(nb, _NCH)):
            start_g(j, j)
        for j in range(_NCH):
            b = j % nb
            wait_g(b)
            start_s(j, b)
            if j + nb < _NCH:
                wait_s(b)
                start_g(j + nb, b)
        for j in range(max(0, _NCH - nb), _NCH):
            wait_s(j % nb)

        plsc.subcore_barrier()
        pltpu.sync_copy(t_sh.at[pl.ds(s * _RPT, _RPT)], zb_v)
        pltpu.sync_copy(zb_v,
                        t_out.at[pl.ds(s * _RPT, _RPT), pl.ds(c * D, D)])

    return pl.kernel(
        body,
        out_type=jax.ShapeDtypeStruct((_NP, 128), jnp.float32),
        mesh=_mesh(),
        scratch_types=[
            pltpu.VMEM((_NCH, _CH), jnp.int32),
            pltpu.VMEM((_NCH, _CH), jnp.int32),
            pltpu.VMEM((_CH, D), jnp.float32),
            pltpu.VMEM((_CH, D), jnp.float32),
            pltpu.VMEM((_CH, D), jnp.float32),
            pltpu.VMEM((_CH, D), jnp.float32),
            pltpu.VMEM((_RPT, D), jnp.float32),
            pltpu.VMEM_SHARED((_NP, D), jnp.float32),
            pltpu.SemaphoreType.DMA,
            pltpu.SemaphoreType.DMA,
            pltpu.SemaphoreType.DMA,
            pltpu.SemaphoreType.DMA,
            pltpu.SemaphoreType.DMA,
            pltpu.SemaphoreType.DMA,
            pltpu.SemaphoreType.DMA,
            pltpu.SemaphoreType.DMA,
        ],
        compiler_params=pltpu.CompilerParams(
            needs_layout_passes=False, use_tc_tiling_on_sc=False
        ),
    )


_BN = 1024
_NB = 10


def _tc_d1_body(x_ref, w10_ref, w11_ref, hist_ref, z0_ref, dis_ref, zs1_ref):
    deg = jnp.sum(hist_ref[...], axis=0, keepdims=True)
    safe = jnp.where(deg > 0, deg, 1.0)
    dis = jnp.where(deg > 0, lax.rsqrt(safe), 0.0)
    dis_col = dis.reshape(dis.shape[1], 1)
    dis_ref[...] = dis_col
    xb = x_ref[...]
    z0_ref[...] = jnp.dot(xb, w10_ref[...], preferred_element_type=jnp.float32)
    zs1_ref[...] = (
        jnp.dot(xb, w11_ref[...], preferred_element_type=jnp.float32) * dis_col
    )


@functools.lru_cache(maxsize=None)
def _tc_d1(din, dh):
    return pl.pallas_call(
        _tc_d1_body,
        grid=(_NB,),
        in_specs=[
            pl.BlockSpec((_BN, din), lambda i: (i, 0)),
            pl.BlockSpec((din, dh), lambda i: (0, 0)),
            pl.BlockSpec((din, dh), lambda i: (0, 0)),
            pl.BlockSpec((_NW, _BN), lambda i: (0, i)),
        ],
        out_specs=[
            pl.BlockSpec((_BN, dh), lambda i: (i, 0)),
            pl.BlockSpec((_BN, 1), lambda i: (i, 0)),
            pl.BlockSpec((_BN, dh), lambda i: (i, 0)),
        ],
        out_shape=[
            jax.ShapeDtypeStruct((_N, dh), jnp.float32),
            jax.ShapeDtypeStruct((_NP, 1), jnp.float32),
            jax.ShapeDtypeStruct((_N, dh), jnp.float32),
        ],
    )


def _tc_d2_body(z0_ref, t1_ref, dis_ref, b1_ref, w20_ref, w21_ref,
                z20_ref, zs2_ref):
    dh = z0_ref.shape[1]
    t = t1_ref[:, :dh] + t1_ref[:, dh:2 * dh]
    dis = dis_ref[...]
    h = jnp.maximum(z0_ref[...] - dis * t + b1_ref[...], 0.0)
    z20_ref[...] = jnp.dot(h, w20_ref[...], preferred_element_type=jnp.float32)
    zs2_ref[...] = (
        jnp.dot(h, w21_ref[...], preferred_element_type=jnp.float32) * dis
    )


@functools.lru_cache(maxsize=None)
def _tc_d2(dh, do):
    return pl.pallas_call(
        _tc_d2_body,
        grid=(_NB,),
        in_specs=[
            pl.BlockSpec((_BN, dh), lambda i: (i, 0)),
            pl.BlockSpec((_BN, 128), lambda i: (i, 0)),
            pl.BlockSpec((_BN, 1), lambda i: (i, 0)),
            pl.BlockSpec((1, dh), lambda i: (0, 0)),
            pl.BlockSpec((dh, do), lambda i: (0, 0)),
            pl.BlockSpec((dh, do), lambda i: (0, 0)),
        ],
        out_specs=[
            pl.BlockSpec((_BN, do), lambda i: (i, 0)),
            pl.BlockSpec((_BN, do), lambda i: (i, 0)),
        ],
        out_shape=[
            jax.ShapeDtypeStruct((_N, do), jnp.float32),
            jax.ShapeDtypeStruct((_N, do), jnp.float32),
        ],
    )


_BNF = 1024
_NBF = 10


def _tc_fin_body(z20_ref, t2_ref, dis_ref, b2_ref, out_ref):
    do = z20_ref.shape[1]
    t = t2_ref[:, :do] + t2_ref[:, do:2 * do]
    o = z20_ref[...] - dis_ref[...] * t + b2_ref[...]
    m = jnp.max(o, axis=1, keepdims=True)
    lse = jnp.log(jnp.sum(jnp.exp(o - m), axis=1, keepdims=True))
    out_ref[...] = o - m - lse


@functools.lru_cache(maxsize=None)
def _tc_fin(do):
    return pl.pallas_call(
        _tc_fin_body,
        grid=(_NBF,),
        in_specs=[
            pl.BlockSpec((_BNF, do), lambda i: (i, 0)),
            pl.BlockSpec((_BNF, 128), lambda i: (i, 0)),
            pl.BlockSpec((_BNF, 1), lambda i: (i, 0)),
            pl.BlockSpec((1, do), lambda i: (0, 0)),
        ],
        out_specs=pl.BlockSpec((_BNF, do), lambda i: (i, 0)),
        out_shape=jax.ShapeDtypeStruct((_N, do), jnp.float32),
    )


def kernel(x, edge_index, W10, W11, b1, W20, W21, b2):
    din = x.shape[1]
    dh = W10.shape[1]
    do = W20.shape[1]

    rowpad, colpad, hist = _sc_pre()(edge_index.reshape(2 * _E))
    z0, dis, zs1 = _tc_d1(din, dh)(x, W10, W11, hist)

    row2d = rowpad.reshape(_NW * _NCH, _CH)
    col2d = colpad.reshape(_NW * _NCH, _CH)
    t1 = _sc_edge(dh)(zs1, row2d, col2d)
    z20, zs2 = _tc_d2(dh, do)(z0, t1, dis, b1.reshape(1, dh), W20, W21)
    t2 = _sc_edge(do)(zs2, row2d, col2d)
    return _tc_fin(do)(z20, t2, dis, b2.reshape(1, do))
